# baseline (speedup 1.0000x reference)
import functools

import jax
import jax.numpy as jnp
from jax import lax
from jax.experimental import pallas as pl
from jax.experimental.pallas import tpu as pltpu
from jax.experimental.pallas import tpu_sc as plsc

N = 10000
E = 160000
D = 256
G = 64
RATIO = 0.8

NP = 10240
NC, NS, L = 2, 16, 16
NW = NC * NS
EP = 163840
EPW = EP // NW
EPT = EP // NS
CH = 128
NCH = EPT // CH
SLAB = NP // NS

_BIG_NEG = -3.4e38


def _deg_body(rows_hbm, cols_hbm, keepf_hbm, zeros_hbm, out_hbm,
              rows_v, cols_v, keepf_v, deg_v):
    cid = lax.axis_index("c")
    sid = lax.axis_index("s")
    wid = sid * NC + cid
    base = wid * EPW
    pltpu.sync_copy(rows_hbm.at[pl.ds(base, EPW)], rows_v)
    pltpu.sync_copy(cols_hbm.at[pl.ds(base, EPW)], cols_v)
    pltpu.sync_copy(keepf_hbm, keepf_v)
    pltpu.sync_copy(zeros_hbm, deg_v)

    def ebody(i, _):
        idx_c = cols_v[pl.ds(i * L, L)]
        idx_r = rows_v[pl.ds(i * L, L)]
        vals = plsc.load_gather(keepf_v, [idx_c])
        plsc.addupdate_scatter(deg_v, [idx_r], vals)
        return 0

    lax.fori_loop(0, EPW // L, ebody, 0)
    pltpu.sync_copy(deg_v, out_hbm.at[wid])


def _sc_degree(rows, cols, keepf, zeros_np):
    mesh = plsc.VectorSubcoreMesh(core_axis_name="c", subcore_axis_name="s")
    f = pl.kernel(
        _deg_body,
        out_type=jax.ShapeDtypeStruct((NW, NP), jnp.float32),
        mesh=mesh,
        compiler_params=pltpu.CompilerParams(needs_layout_passes=False),
        scratch_types=[
            pltpu.VMEM((EPW,), jnp.int32),
            pltpu.VMEM((EPW,), jnp.int32),
            pltpu.VMEM((NP,), jnp.float32),
            pltpu.VMEM((NP,), jnp.float32),
        ],
    )
    return f(rows, cols, keepf, zeros_np)


def _msg_half(hs_hbm, out_hbm, cid, sid, ridx_v, cidx_v, buf2, acc_sh, sems):
    def chunk(g, _):
        cp = pltpu.make_async_copy(hs_hbm.at[ridx_v.at[g]], buf2, sems)
        cp.start()
        cp.wait()
        pltpu.sync_copy(buf2, acc_sh.at[cidx_v.at[g]], add=True)
        return 0

    lax.fori_loop(0, NCH, chunk, 0)
    plsc.subcore_barrier()

    def wb(j, _):
        r0 = sid * SLAB + j * CH
        pltpu.sync_copy(acc_sh.at[pl.ds(r0, CH)], buf2)
        pltpu.sync_copy(buf2, out_hbm.at[pl.ds(r0, CH)])
        return 0

    lax.fori_loop(0, SLAB // CH, wb, 0)


def _msg_body2(rows3_hbm, cols3_hbm, hlo_hbm, hhi_hbm, zeros_hbm,
               outlo_hbm, outhi_hbm, ridx_v, cidx_v, buf2, acc_sh, sems):
    cid = lax.axis_index("c")
    sid = lax.axis_index("s")
    pltpu.sync_copy(rows3_hbm.at[sid], ridx_v)
    pltpu.sync_copy(cols3_hbm.at[sid], cidx_v)
    r0 = sid * SLAB
    pltpu.sync_copy(zeros_hbm.at[pl.ds(r0, SLAB)], acc_sh.at[pl.ds(r0, SLAB)])
    plsc.subcore_barrier()

    @pl.when(cid == 0)
    def _():
        _msg_half(hlo_hbm, outlo_hbm, cid, sid, ridx_v, cidx_v,
                  buf2, acc_sh, sems)

    @pl.when(cid == 1)
    def _():
        _msg_half(hhi_hbm, outhi_hbm, cid, sid, ridx_v, cidx_v,
                  buf2, acc_sh, sems)


def _sc_messages2(rows3, cols3, hs_lo, hs_hi, zeros_half):
    mesh = plsc.VectorSubcoreMesh(core_axis_name="c", subcore_axis_name="s")
    f = pl.kernel(
        _msg_body2,
        out_type=(jax.ShapeDtypeStruct((NP, 128), jnp.float32),
                  jax.ShapeDtypeStruct((NP, 128), jnp.float32)),
        mesh=mesh,
        compiler_params=pltpu.CompilerParams(needs_layout_passes=False),
        scratch_types=[
            pltpu.VMEM((NCH, CH), jnp.int32),
            pltpu.VMEM((NCH, CH), jnp.int32),
            pltpu.VMEM((CH, 128), jnp.float32),
            pltpu.VMEM_SHARED((NP, 128), jnp.float32),
            pltpu.SemaphoreType.DMA,
        ],
    )
    return f(rows3, cols3, hs_lo, hs_hi, zeros_half)


def _lin_body(hlo_ref, hhi_ref, W_ref, b_ref, valid_ref, degp_ref,
              hslo_ref, hshi_ref, dis_ref):
    deg = jnp.sum(degp_ref[...], axis=1, keepdims=True) + valid_ref[...]
    dis = valid_ref[...] * lax.rsqrt(jnp.maximum(deg, jnp.float32(1e-30)))
    dis_ref[...] = dis
    W = W_ref[...]
    wlo = W[:, :128]
    whi = W[:, 128:]
    hlin = (lax.dot_general(hlo_ref[...], wlo, (((1,), (1,)), ((), ())),
                            preferred_element_type=jnp.float32)
            + lax.dot_general(hhi_ref[...], whi, (((1,), (1,)), ((), ())),
                              preferred_element_type=jnp.float32)
            + b_ref[...])
    hs = hlin * dis
    hslo_ref[...] = hs[:, :128]
    hshi_ref[...] = hs[:, 128:]


def _tc_linear(h_lo, h_hi, W, b2, valid_c, degp_t):
    BR = 1024
    nblk = NP // BR
    grid = (nblk,)
    f = pl.pallas_call(
        _lin_body,
        grid=grid,
        in_specs=[
            pl.BlockSpec((BR, 128), lambda i: (i, 0)),
            pl.BlockSpec((BR, 128), lambda i: (i, 0)),
            pl.BlockSpec((D, D), lambda i: (0, 0)),
            pl.BlockSpec((1, D), lambda i: (0, 0)),
            pl.BlockSpec((BR, 1), lambda i: (i, 0)),
            pl.BlockSpec((BR, NW), lambda i: (i, 0)),
        ],
        out_specs=[
            pl.BlockSpec((BR, 128), lambda i: (i, 0)),
            pl.BlockSpec((BR, 128), lambda i: (i, 0)),
            pl.BlockSpec((BR, 1), lambda i: (i, 0)),
        ],
        out_shape=[
            jax.ShapeDtypeStruct((NP, 128), jnp.float32),
            jax.ShapeDtypeStruct((NP, 128), jnp.float32),
            jax.ShapeDtypeStruct((NP, 1), jnp.float32),
        ],
    )
    return f(h_lo, h_hi, W, b2, valid_c, degp_t)


def _score_body(acclo_ref, acchi_ref, hslo_ref, hshi_ref, dis_ref,
                plo_ref, phi_ref, hrlo_ref, hrhi_ref, score_ref):
    dis = dis_ref[...]
    hrel_lo = jnp.maximum(dis * (acclo_ref[...] + hslo_ref[...]), 0.0)
    hrel_hi = jnp.maximum(dis * (acchi_ref[...] + hshi_ref[...]), 0.0)
    hrlo_ref[...] = hrel_lo
    hrhi_ref[...] = hrel_hi
    plo = plo_ref[...]
    phi = phi_ref[...]
    pnorm = lax.rsqrt(jnp.sum(plo * plo) + jnp.sum(phi * phi))
    sc = (lax.dot_general(hrel_lo, plo, (((1,), (0,)), ((), ())),
                          preferred_element_type=jnp.float32)
          + lax.dot_general(hrel_hi, phi, (((1,), (0,)), ((), ())),
                            preferred_element_type=jnp.float32))
    score_ref[...] = jnp.tanh(sc * pnorm)


def _tc_score(acc_lo, acc_hi, hs_lo, hs_hi, dis_c, p_lo, p_hi):
    BR = 2048
    f = pl.pallas_call(
        _score_body,
        grid=(NP // BR,),
        in_specs=[
            pl.BlockSpec((BR, 128), lambda i: (i, 0)),
            pl.BlockSpec((BR, 128), lambda i: (i, 0)),
            pl.BlockSpec((BR, 128), lambda i: (i, 0)),
            pl.BlockSpec((BR, 128), lambda i: (i, 0)),
            pl.BlockSpec((BR, 1), lambda i: (i, 0)),
            pl.BlockSpec((128, 1), lambda i: (0, 0)),
            pl.BlockSpec((128, 1), lambda i: (0, 0)),
        ],
        out_specs=[
            pl.BlockSpec((BR, 128), lambda i: (i, 0)),
            pl.BlockSpec((BR, 128), lambda i: (i, 0)),
            pl.BlockSpec((BR, 1), lambda i: (i, 0)),
        ],
        out_shape=[
            jax.ShapeDtypeStruct((NP, 128), jnp.float32),
            jax.ShapeDtypeStruct((NP, 128), jnp.float32),
            jax.ShapeDtypeStruct((NP, 1), jnp.float32),
        ],
    )
    return f(acc_lo, acc_hi, hs_lo, hs_hi, dis_c, p_lo, p_hi)


def _topk_body(score_ref, valid_ref, oh_ref, keep_ref):
    score = score_ref[...]
    b = lax.bitcast_convert_type(score, jnp.int32)
    minint = jnp.int32(-2147483648)
    key = jnp.where(b < 0, minint - b, b)

    valid = valid_ref[...]
    ohv = oh_ref[...] * valid
    counts = jnp.sum(ohv, axis=0, keepdims=True)
    kf = jnp.ceil(jnp.float32(RATIO) * counts)

    def bit_step(i, T):
        cand = T + lax.shift_left(jnp.int32(1), jnp.int32(31) - i)
        ge = (key >= cand).astype(jnp.float32)
        cnt = jnp.sum(ge * ohv, axis=0, keepdims=True)
        return jnp.where(cnt >= kf, cand, T)

    T0 = jnp.full((1, G), minint, jnp.int32)
    T = lax.fori_loop(0, 32, bit_step, T0)
    ge_fin = (key >= T).astype(jnp.float32)
    keep_ref[...] = (jnp.sum(ge_fin * ohv, axis=1, keepdims=True)
                     > 0.0).astype(jnp.float32)


def _tc_topk(score_c, valid_c, oh):
    f = pl.pallas_call(
        _topk_body,
        out_shape=jax.ShapeDtypeStruct((NP, 1), jnp.float32),
    )
    return f(score_c, valid_c, oh)


def _mask_body(hrlo_ref, hrhi_ref, keep_ref, score_ref, hnlo_ref, hnhi_ref):
    m = keep_ref[...] * score_ref[...]
    hnlo_ref[...] = m * hrlo_ref[...]
    hnhi_ref[...] = m * hrhi_ref[...]


def _tc_mask(hrel_lo, hrel_hi, keep_c, score_c):
    BR = 2048
    f = pl.pallas_call(
        _mask_body,
        grid=(NP // BR,),
        in_specs=[
            pl.BlockSpec((BR, 128), lambda i: (i, 0)),
            pl.BlockSpec((BR, 128), lambda i: (i, 0)),
            pl.BlockSpec((BR, 1), lambda i: (i, 0)),
            pl.BlockSpec((BR, 1), lambda i: (i, 0)),
        ],
        out_specs=[
            pl.BlockSpec((BR, 128), lambda i: (i, 0)),
            pl.BlockSpec((BR, 128), lambda i: (i, 0)),
        ],
        out_shape=[
            jax.ShapeDtypeStruct((NP, 128), jnp.float32),
            jax.ShapeDtypeStruct((NP, 128), jnp.float32),
        ],
    )
    return f(hrel_lo, hrel_hi, keep_c, score_c)


def _feats_body(starts_ref, hnlo_ref, hnhi_ref, keep_ref,
                maxlo_ref, maxhi_ref, sumlo_ref, sumhi_ref, cnt_ref):
    g = pl.program_id(0)
    start = starts_ref[g]
    end = starts_ref[g + 1]
    nb = (end - start + 7) // 8

    def body(i, carry):
        mlo, mhi, slo, shi, c = carry
        r0 = start + i * 8
        pos = r0 + lax.broadcasted_iota(jnp.int32, (8, 1), 0)
        inseg = (pos < end).astype(jnp.float32)
        kp = keep_ref[pl.ds(r0, 8), :] * inseg
        rl = hnlo_ref[pl.ds(r0, 8), :]
        rh = hnhi_ref[pl.ds(r0, 8), :]
        mlo = jnp.maximum(mlo, jnp.where(kp > 0, rl, _BIG_NEG))
        mhi = jnp.maximum(mhi, jnp.where(kp > 0, rh, _BIG_NEG))
        slo = slo + rl * kp
        shi = shi + rh * kp
        c = c + jnp.sum(kp)
        return mlo, mhi, slo, shi, c

    init = (jnp.full((8, 128), _BIG_NEG, jnp.float32),
            jnp.full((8, 128), _BIG_NEG, jnp.float32),
            jnp.zeros((8, 128), jnp.float32),
            jnp.zeros((8, 128), jnp.float32),
            jnp.float32(0.0))
    mlo, mhi, slo, shi, c = lax.fori_loop(0, nb, body, init)
    maxlo_ref[...] = jnp.broadcast_to(
        jnp.max(mlo, axis=0, keepdims=True), (8, 128)).reshape(1, 8, 128)
    maxhi_ref[...] = jnp.broadcast_to(
        jnp.max(mhi, axis=0, keepdims=True), (8, 128)).reshape(1, 8, 128)
    sumlo_ref[...] = jnp.broadcast_to(
        jnp.sum(slo, axis=0, keepdims=True), (8, 128)).reshape(1, 8, 128)
    sumhi_ref[...] = jnp.broadcast_to(
        jnp.sum(shi, axis=0, keepdims=True), (8, 128)).reshape(1, 8, 128)
    cnt_ref[...] = jnp.full((1, 8, 1), c, jnp.float32)


def _tc_feats(starts, hn_lo, hn_hi, keep_c):
    grid_spec = pltpu.PrefetchScalarGridSpec(
        num_scalar_prefetch=1,
        grid=(G,),
        in_specs=[
            pl.BlockSpec((NP, 128), lambda g, s: (0, 0)),
            pl.BlockSpec((NP, 128), lambda g, s: (0, 0)),
            pl.BlockSpec((NP, 1), lambda g, s: (0, 0)),
        ],
        out_specs=[
            pl.BlockSpec((1, 8, 128), lambda g, s: (g, 0, 0)),
            pl.BlockSpec((1, 8, 128), lambda g, s: (g, 0, 0)),
            pl.BlockSpec((1, 8, 128), lambda g, s: (g, 0, 0)),
            pl.BlockSpec((1, 8, 128), lambda g, s: (g, 0, 0)),
            pl.BlockSpec((1, 8, 1), lambda g, s: (g, 0, 0)),
        ],
    )
    f = pl.pallas_call(
        _feats_body,
        grid_spec=grid_spec,
        out_shape=[
            jax.ShapeDtypeStruct((G, 8, 128), jnp.float32),
            jax.ShapeDtypeStruct((G, 8, 128), jnp.float32),
            jax.ShapeDtypeStruct((G, 8, 128), jnp.float32),
            jax.ShapeDtypeStruct((G, 8, 128), jnp.float32),
            jax.ShapeDtypeStruct((G, 8, 1), jnp.float32),
        ],
    )
    mlo, mhi, slo, shi, cnt = f(starts, hn_lo, hn_hi, keep_c)
    return mlo[:, 0, :], mhi[:, 0, :], slo[:, 0, :], shi[:, 0, :], cnt[:, 0, :]


def _head_body(*refs):
    feats_refs = refs[0:15]
    mlp_refs = refs[15:27]
    head_refs = refs[27:33]
    out_ref = refs[33]

    sx = None
    for li in range(3):
        mlo, mhi, slo, shi, cnt = feats_refs[li * 5:(li + 1) * 5]
        cdiv = jnp.maximum(cnt[...], 1.0)
        f = jnp.concatenate(
            [mlo[...], mhi[...], slo[...] / cdiv, shi[...] / cdiv], axis=1)
        sx = f if sx is None else sx + f
    sx = sx * jnp.float32(1.0 / 3.0)

    a = sx
    for li in range(6):
        W = mlp_refs[li * 2][...]
        b = mlp_refs[li * 2 + 1][...]
        a = jnp.maximum(
            lax.dot_general(a, W, (((1,), (1,)), ((), ())),
                            preferred_element_type=jnp.float32) + b, 0.0)
    z = (a + 1.0) * sx
    for li in range(3):
        W = head_refs[li * 2][...]
        b = head_refs[li * 2 + 1][...]
        z = lax.dot_general(z, W, (((1,), (1,)), ((), ())),
                            preferred_element_type=jnp.float32) + b
        if li < 2:
            z = jnp.maximum(z, 0.0)
    m = jnp.max(z, axis=1, keepdims=True)
    e = z - m
    lse = jnp.log(jnp.sum(jnp.exp(e), axis=1, keepdims=True))
    out_ref[...] = e - lse


def _tc_head(feats_list, mlp_list, head_list):
    args = list(feats_list) + list(mlp_list) + list(head_list)
    f = pl.pallas_call(
        _head_body,
        out_shape=jax.ShapeDtypeStruct((G, 2), jnp.float32),
    )
    return f(*args)


def kernel(x, edge_index, batch, W1, b1, W2, b2, W3, b3, p1, p2, p3,
           aW1, ab1, aW2, ab2, aW3, ab3, aW4, ab4, aW5, ab5, aW6, ab6,
           lW1, lb1, lW2, lb2, lW3, lb3):
    ei = edge_index.astype(jnp.int32)
    batch_i = batch.astype(jnp.int32)

    pad_e = EP - E
    rows = jnp.concatenate([ei[0], jnp.full((pad_e,), N, jnp.int32)])
    cols = jnp.concatenate([ei[1], jnp.full((pad_e,), N, jnp.int32)])
    rows3 = rows.reshape(NS, NCH, CH)
    cols3 = cols.reshape(NS, NCH, CH)

    xp = jnp.pad(x, ((0, NP - N), (0, 0)))
    h_lo = xp[:, :128]
    h_hi = xp[:, 128:]
    validf = jnp.pad(jnp.ones((N,), jnp.float32), (0, NP - N))
    valid_c = validf.reshape(NP, 1)
    oh = (batch_i[:, None] == jnp.arange(G, dtype=jnp.int32)[None, :])
    oh = jnp.pad(oh.astype(jnp.float32), ((0, NP - N), (0, 0)))
    starts = jnp.searchsorted(batch_i, jnp.arange(G + 1, dtype=jnp.int32),
                              side="left").astype(jnp.int32)

    zeros_np = jnp.zeros((NP,), jnp.float32)
    zeros_half = jnp.zeros((NP, 128), jnp.float32)

    layer_params = [(W1, b1, p1), (W2, b2, p2), (W3, b3, p3)]
    feats = []
    keepf = validf
    for (W, b, p) in layer_params:
        degp = _sc_degree(rows, cols, keepf, zeros_np)
        degp_t = degp.T
        hs_lo, hs_hi, dis_c = _tc_linear(
            h_lo, h_hi, W, b.reshape(1, D), valid_c, degp_t)
        acc_lo, acc_hi = _sc_messages2(rows3, cols3, hs_lo, hs_hi, zeros_half)
        hrel_lo, hrel_hi, score_c = _tc_score(
            acc_lo, acc_hi, hs_lo, hs_hi, dis_c,
            p[:128].reshape(128, 1), p[128:].reshape(128, 1))
        keep_c = _tc_topk(score_c, valid_c, oh)
        hn_lo, hn_hi = _tc_mask(hrel_lo, hrel_hi, keep_c, score_c)
        mlo, mhi, slo, shi, cnt = _tc_feats(starts, hn_lo, hn_hi, keep_c)
        feats.extend([mlo, mhi, slo, shi, cnt])
        h_lo, h_hi = hn_lo, hn_hi
        keepf = keep_c.reshape(NP)
        valid_c = keep_c

    mlp_list = [aW1, ab1.reshape(1, -1), aW2, ab2.reshape(1, -1),
                aW3, ab3.reshape(1, -1), aW4, ab4.reshape(1, -1),
                aW5, ab5.reshape(1, -1), aW6, ab6.reshape(1, -1)]
    head_list = [lW1, lb1.reshape(1, -1), lW2, lb2.reshape(1, -1),
                 lW3, lb3.reshape(1, -1)]
    return _tc_head(feats, mlp_list, head_list)


# speedup vs baseline: 1.0693x; 1.0693x over previous
import functools

import jax
import jax.numpy as jnp
from jax import lax
from jax.experimental import pallas as pl
from jax.experimental.pallas import tpu as pltpu
from jax.experimental.pallas import tpu_sc as plsc

N = 10000
E = 160000
D = 256
G = 64
RATIO = 0.8

NP = 10240
NC, NS, L = 2, 16, 16
NW = NC * NS
EP = 163840
EPW = EP // NW
EPT = EP // NS
CH = 128
NCH = EPT // CH
SLAB = NP // NS

_BIG_NEG = -3.4e38


def _deg_body(rows_hbm, cols_hbm, keepf_hbm, zeros_hbm, out_hbm,
              rows_v, cols_v, keepf_v, deg_v):
    cid = lax.axis_index("c")
    sid = lax.axis_index("s")
    wid = sid * NC + cid
    base = wid * EPW
    pltpu.sync_copy(rows_hbm.at[pl.ds(base, EPW)], rows_v)
    pltpu.sync_copy(cols_hbm.at[pl.ds(base, EPW)], cols_v)
    pltpu.sync_copy(keepf_hbm, keepf_v)
    pltpu.sync_copy(zeros_hbm, deg_v)

    def ebody(i, _):
        idx_c = cols_v[pl.ds(i * L, L)]
        idx_r = rows_v[pl.ds(i * L, L)]
        vals = plsc.load_gather(keepf_v, [idx_c])
        plsc.addupdate_scatter(deg_v, [idx_r], vals)
        return 0

    lax.fori_loop(0, EPW // L, ebody, 0)
    pltpu.sync_copy(deg_v, out_hbm.at[wid])


def _sc_degree(rows, cols, keepf, zeros_np):
    mesh = plsc.VectorSubcoreMesh(core_axis_name="c", subcore_axis_name="s")
    f = pl.kernel(
        _deg_body,
        out_type=jax.ShapeDtypeStruct((NW, NP), jnp.float32),
        mesh=mesh,
        compiler_params=pltpu.CompilerParams(needs_layout_passes=False),
        scratch_types=[
            pltpu.VMEM((EPW,), jnp.int32),
            pltpu.VMEM((EPW,), jnp.int32),
            pltpu.VMEM((NP,), jnp.float32),
            pltpu.VMEM((NP,), jnp.float32),
        ],
    )
    return f(rows, cols, keepf, zeros_np)


RB = 4


def _msg_half(hs_hbm, rows3_hbm, cols3_hbm, out_hbm, sid,
              rring, cring, buf2, acc_sh, gsem, rsem, csem):
    def idx_fetch(g):
        s = lax.rem(g, RB)
        pltpu.make_async_copy(rows3_hbm.at[sid, g], rring.at[s],
                              rsem.at[s]).start()
        pltpu.make_async_copy(cols3_hbm.at[sid, g], cring.at[s],
                              csem.at[s]).start()

    idx_fetch(0)
    idx_fetch(1)

    def chunk(g, _):
        s = lax.rem(g, RB)
        pltpu.make_async_copy(rows3_hbm.at[sid, 0], rring.at[s],
                              rsem.at[s]).wait()
        pltpu.make_async_copy(cols3_hbm.at[sid, 0], cring.at[s],
                              csem.at[s]).wait()
        p = lax.rem(g, 2)
        pltpu.make_async_copy(hs_hbm.at[rring.at[s]], buf2.at[p],
                              gsem.at[p]).start()

        @pl.when(g + 2 < NCH)
        def _():
            idx_fetch(g + 2)

        @pl.when(g > 0)
        def _():
            q = lax.rem(g + 1, 2)
            s1 = lax.rem(g + RB - 1, RB)
            pltpu.make_async_copy(hs_hbm.at[pl.ds(0, CH)], buf2.at[q],
                                  gsem.at[q]).wait()
            pltpu.sync_copy(buf2.at[q], acc_sh.at[cring.at[s1]], add=True)
        return 0

    lax.fori_loop(0, NCH, chunk, 0)
    qf = (NCH - 1) % 2
    sf = (NCH - 1) % RB
    pltpu.make_async_copy(hs_hbm.at[pl.ds(0, CH)], buf2.at[qf],
                          gsem.at[qf]).wait()
    pltpu.sync_copy(buf2.at[qf], acc_sh.at[cring.at[sf]], add=True)
    plsc.subcore_barrier()

    def wb(j, _):
        r0 = sid * SLAB + j * CH
        pltpu.sync_copy(acc_sh.at[pl.ds(r0, CH)], buf2.at[0])
        pltpu.sync_copy(buf2.at[0], out_hbm.at[pl.ds(r0, CH)])
        return 0

    lax.fori_loop(0, SLAB // CH, wb, 0)


def _msg_body2(rows3_hbm, cols3_hbm, hlo_hbm, hhi_hbm, zeros_hbm,
               outlo_hbm, outhi_hbm, rring, cring, buf2, acc_sh,
               gsem, rsem, csem):
    cid = lax.axis_index("c")
    sid = lax.axis_index("s")
    r0 = sid * SLAB
    pltpu.sync_copy(zeros_hbm.at[pl.ds(r0, SLAB)], acc_sh.at[pl.ds(r0, SLAB)])
    plsc.subcore_barrier()

    @pl.when(cid == 0)
    def _():
        _msg_half(hlo_hbm, rows3_hbm, cols3_hbm, outlo_hbm, sid,
                  rring, cring, buf2, acc_sh, gsem, rsem, csem)

    @pl.when(cid == 1)
    def _():
        _msg_half(hhi_hbm, rows3_hbm, cols3_hbm, outhi_hbm, sid,
                  rring, cring, buf2, acc_sh, gsem, rsem, csem)


def _sc_messages2(rows3, cols3, hs_lo, hs_hi, zeros_half):
    mesh = plsc.VectorSubcoreMesh(core_axis_name="c", subcore_axis_name="s")
    f = pl.kernel(
        _msg_body2,
        out_type=(jax.ShapeDtypeStruct((NP, 128), jnp.float32),
                  jax.ShapeDtypeStruct((NP, 128), jnp.float32)),
        mesh=mesh,
        compiler_params=pltpu.CompilerParams(needs_layout_passes=False),
        scratch_types=[
            pltpu.VMEM((RB, CH), jnp.int32),
            pltpu.VMEM((RB, CH), jnp.int32),
            pltpu.VMEM((2, CH, 128), jnp.float32),
            pltpu.VMEM_SHARED((NP, 128), jnp.float32),
            pltpu.SemaphoreType.DMA((2,)),
            pltpu.SemaphoreType.DMA((RB,)),
            pltpu.SemaphoreType.DMA((RB,)),
        ],
    )
    return f(rows3, cols3, hs_lo, hs_hi, zeros_half)


def _lin_body(hlo_ref, hhi_ref, W_ref, b_ref, valid_ref, degp_ref,
              hslo_ref, hshi_ref, dis_ref):
    deg = jnp.sum(degp_ref[...], axis=1, keepdims=True) + valid_ref[...]
    dis = valid_ref[...] * lax.rsqrt(jnp.maximum(deg, jnp.float32(1e-30)))
    dis_ref[...] = dis
    W = W_ref[...]
    wlo = W[:, :128]
    whi = W[:, 128:]
    hlin = (lax.dot_general(hlo_ref[...], wlo, (((1,), (1,)), ((), ())),
                            preferred_element_type=jnp.float32)
            + lax.dot_general(hhi_ref[...], whi, (((1,), (1,)), ((), ())),
                              preferred_element_type=jnp.float32)
            + b_ref[...])
    hs = hlin * dis
    hslo_ref[...] = hs[:, :128]
    hshi_ref[...] = hs[:, 128:]


def _tc_linear(h_lo, h_hi, W, b2, valid_c, degp_t):
    BR = 1024
    nblk = NP // BR
    grid = (nblk,)
    f = pl.pallas_call(
        _lin_body,
        grid=grid,
        in_specs=[
            pl.BlockSpec((BR, 128), lambda i: (i, 0)),
            pl.BlockSpec((BR, 128), lambda i: (i, 0)),
            pl.BlockSpec((D, D), lambda i: (0, 0)),
            pl.BlockSpec((1, D), lambda i: (0, 0)),
            pl.BlockSpec((BR, 1), lambda i: (i, 0)),
            pl.BlockSpec((BR, NW), lambda i: (i, 0)),
        ],
        out_specs=[
            pl.BlockSpec((BR, 128), lambda i: (i, 0)),
            pl.BlockSpec((BR, 128), lambda i: (i, 0)),
            pl.BlockSpec((BR, 1), lambda i: (i, 0)),
        ],
        out_shape=[
            jax.ShapeDtypeStruct((NP, 128), jnp.float32),
            jax.ShapeDtypeStruct((NP, 128), jnp.float32),
            jax.ShapeDtypeStruct((NP, 1), jnp.float32),
        ],
    )
    return f(h_lo, h_hi, W, b2, valid_c, degp_t)


def _score_body(acclo_ref, acchi_ref, hslo_ref, hshi_ref, dis_ref,
                plo_ref, phi_ref, hrlo_ref, hrhi_ref, score_ref):
    dis = dis_ref[...]
    hrel_lo = jnp.maximum(dis * (acclo_ref[...] + hslo_ref[...]), 0.0)
    hrel_hi = jnp.maximum(dis * (acchi_ref[...] + hshi_ref[...]), 0.0)
    hrlo_ref[...] = hrel_lo
    hrhi_ref[...] = hrel_hi
    plo = plo_ref[...]
    phi = phi_ref[...]
    pnorm = lax.rsqrt(jnp.sum(plo * plo) + jnp.sum(phi * phi))
    sc = (lax.dot_general(hrel_lo, plo, (((1,), (0,)), ((), ())),
                          preferred_element_type=jnp.float32)
          + lax.dot_general(hrel_hi, phi, (((1,), (0,)), ((), ())),
                            preferred_element_type=jnp.float32))
    score_ref[...] = jnp.tanh(sc * pnorm)


def _tc_score(acc_lo, acc_hi, hs_lo, hs_hi, dis_c, p_lo, p_hi):
    BR = 2048
    f = pl.pallas_call(
        _score_body,
        grid=(NP // BR,),
        in_specs=[
            pl.BlockSpec((BR, 128), lambda i: (i, 0)),
            pl.BlockSpec((BR, 128), lambda i: (i, 0)),
            pl.BlockSpec((BR, 128), lambda i: (i, 0)),
            pl.BlockSpec((BR, 128), lambda i: (i, 0)),
            pl.BlockSpec((BR, 1), lambda i: (i, 0)),
            pl.BlockSpec((128, 1), lambda i: (0, 0)),
            pl.BlockSpec((128, 1), lambda i: (0, 0)),
        ],
        out_specs=[
            pl.BlockSpec((BR, 128), lambda i: (i, 0)),
            pl.BlockSpec((BR, 128), lambda i: (i, 0)),
            pl.BlockSpec((BR, 1), lambda i: (i, 0)),
        ],
        out_shape=[
            jax.ShapeDtypeStruct((NP, 128), jnp.float32),
            jax.ShapeDtypeStruct((NP, 128), jnp.float32),
            jax.ShapeDtypeStruct((NP, 1), jnp.float32),
        ],
    )
    return f(acc_lo, acc_hi, hs_lo, hs_hi, dis_c, p_lo, p_hi)


def _topk_body(score_ref, valid_ref, oh_ref, keep_ref):
    score = score_ref[...]
    b = lax.bitcast_convert_type(score, jnp.int32)
    minint = jnp.int32(-2147483648)
    key = jnp.where(b < 0, minint - b, b)

    valid = valid_ref[...]
    ohv = oh_ref[...] * valid
    counts = jnp.sum(ohv, axis=0, keepdims=True)
    kf = jnp.ceil(jnp.float32(RATIO) * counts)

    def bit_step(i, T):
        cand = T + lax.shift_left(jnp.int32(1), jnp.int32(31) - i)
        ge = (key >= cand).astype(jnp.float32)
        cnt = jnp.sum(ge * ohv, axis=0, keepdims=True)
        return jnp.where(cnt >= kf, cand, T)

    T0 = jnp.full((1, G), minint, jnp.int32)
    T = lax.fori_loop(0, 32, bit_step, T0)
    ge_fin = (key >= T).astype(jnp.float32)
    keep_ref[...] = (jnp.sum(ge_fin * ohv, axis=1, keepdims=True)
                     > 0.0).astype(jnp.float32)


def _tc_topk(score_c, valid_c, oh):
    f = pl.pallas_call(
        _topk_body,
        out_shape=jax.ShapeDtypeStruct((NP, 1), jnp.float32),
    )
    return f(score_c, valid_c, oh)


def _mask_body(hrlo_ref, hrhi_ref, keep_ref, score_ref, hnlo_ref, hnhi_ref):
    m = keep_ref[...] * score_ref[...]
    hnlo_ref[...] = m * hrlo_ref[...]
    hnhi_ref[...] = m * hrhi_ref[...]


def _tc_mask(hrel_lo, hrel_hi, keep_c, score_c):
    BR = 2048
    f = pl.pallas_call(
        _mask_body,
        grid=(NP // BR,),
        in_specs=[
            pl.BlockSpec((BR, 128), lambda i: (i, 0)),
            pl.BlockSpec((BR, 128), lambda i: (i, 0)),
            pl.BlockSpec((BR, 1), lambda i: (i, 0)),
            pl.BlockSpec((BR, 1), lambda i: (i, 0)),
        ],
        out_specs=[
            pl.BlockSpec((BR, 128), lambda i: (i, 0)),
            pl.BlockSpec((BR, 128), lambda i: (i, 0)),
        ],
        out_shape=[
            jax.ShapeDtypeStruct((NP, 128), jnp.float32),
            jax.ShapeDtypeStruct((NP, 128), jnp.float32),
        ],
    )
    return f(hrel_lo, hrel_hi, keep_c, score_c)


def _feats_body(starts_ref, hnlo_ref, hnhi_ref, keep_ref,
                maxlo_ref, maxhi_ref, sumlo_ref, sumhi_ref, cnt_ref):
    g = pl.program_id(0)
    start = starts_ref[g]
    end = starts_ref[g + 1]
    nb = (end - start + 7) // 8

    def body(i, carry):
        mlo, mhi, slo, shi, c = carry
        r0 = start + i * 8
        pos = r0 + lax.broadcasted_iota(jnp.int32, (8, 1), 0)
        inseg = (pos < end).astype(jnp.float32)
        kp = keep_ref[pl.ds(r0, 8), :] * inseg
        rl = hnlo_ref[pl.ds(r0, 8), :]
        rh = hnhi_ref[pl.ds(r0, 8), :]
        mlo = jnp.maximum(mlo, jnp.where(kp > 0, rl, _BIG_NEG))
        mhi = jnp.maximum(mhi, jnp.where(kp > 0, rh, _BIG_NEG))
        slo = slo + rl * kp
        shi = shi + rh * kp
        c = c + jnp.sum(kp)
        return mlo, mhi, slo, shi, c

    init = (jnp.full((8, 128), _BIG_NEG, jnp.float32),
            jnp.full((8, 128), _BIG_NEG, jnp.float32),
            jnp.zeros((8, 128), jnp.float32),
            jnp.zeros((8, 128), jnp.float32),
            jnp.float32(0.0))
    mlo, mhi, slo, shi, c = lax.fori_loop(0, nb, body, init)
    maxlo_ref[...] = jnp.broadcast_to(
        jnp.max(mlo, axis=0, keepdims=True), (8, 128)).reshape(1, 8, 128)
    maxhi_ref[...] = jnp.broadcast_to(
        jnp.max(mhi, axis=0, keepdims=True), (8, 128)).reshape(1, 8, 128)
    sumlo_ref[...] = jnp.broadcast_to(
        jnp.sum(slo, axis=0, keepdims=True), (8, 128)).reshape(1, 8, 128)
    sumhi_ref[...] = jnp.broadcast_to(
        jnp.sum(shi, axis=0, keepdims=True), (8, 128)).reshape(1, 8, 128)
    cnt_ref[...] = jnp.full((1, 8, 1), c, jnp.float32)


def _tc_feats(starts, hn_lo, hn_hi, keep_c):
    grid_spec = pltpu.PrefetchScalarGridSpec(
        num_scalar_prefetch=1,
        grid=(G,),
        in_specs=[
            pl.BlockSpec((NP, 128), lambda g, s: (0, 0)),
            pl.BlockSpec((NP, 128), lambda g, s: (0, 0)),
            pl.BlockSpec((NP, 1), lambda g, s: (0, 0)),
        ],
        out_specs=[
            pl.BlockSpec((1, 8, 128), lambda g, s: (g, 0, 0)),
            pl.BlockSpec((1, 8, 128), lambda g, s: (g, 0, 0)),
            pl.BlockSpec((1, 8, 128), lambda g, s: (g, 0, 0)),
            pl.BlockSpec((1, 8, 128), lambda g, s: (g, 0, 0)),
            pl.BlockSpec((1, 8, 1), lambda g, s: (g, 0, 0)),
        ],
    )
    f = pl.pallas_call(
        _feats_body,
        grid_spec=grid_spec,
        out_shape=[
            jax.ShapeDtypeStruct((G, 8, 128), jnp.float32),
            jax.ShapeDtypeStruct((G, 8, 128), jnp.float32),
            jax.ShapeDtypeStruct((G, 8, 128), jnp.float32),
            jax.ShapeDtypeStruct((G, 8, 128), jnp.float32),
            jax.ShapeDtypeStruct((G, 8, 1), jnp.float32),
        ],
    )
    mlo, mhi, slo, shi, cnt = f(starts, hn_lo, hn_hi, keep_c)
    return mlo[:, 0, :], mhi[:, 0, :], slo[:, 0, :], shi[:, 0, :], cnt[:, 0, :]


def _head_body(*refs):
    feats_refs = refs[0:15]
    mlp_refs = refs[15:27]
    head_refs = refs[27:33]
    out_ref = refs[33]

    sx = None
    for li in range(3):
        mlo, mhi, slo, shi, cnt = feats_refs[li * 5:(li + 1) * 5]
        cdiv = jnp.maximum(cnt[...], 1.0)
        f = jnp.concatenate(
            [mlo[...], mhi[...], slo[...] / cdiv, shi[...] / cdiv], axis=1)
        sx = f if sx is None else sx + f
    sx = sx * jnp.float32(1.0 / 3.0)

    a = sx
    for li in range(6):
        W = mlp_refs[li * 2][...]
        b = mlp_refs[li * 2 + 1][...]
        a = jnp.maximum(
            lax.dot_general(a, W, (((1,), (1,)), ((), ())),
                            preferred_element_type=jnp.float32) + b, 0.0)
    z = (a + 1.0) * sx
    for li in range(3):
        W = head_refs[li * 2][...]
        b = head_refs[li * 2 + 1][...]
        z = lax.dot_general(z, W, (((1,), (1,)), ((), ())),
                            preferred_element_type=jnp.float32) + b
        if li < 2:
            z = jnp.maximum(z, 0.0)
    m = jnp.max(z, axis=1, keepdims=True)
    e = z - m
    lse = jnp.log(jnp.sum(jnp.exp(e), axis=1, keepdims=True))
    out_ref[...] = e - lse


def _tc_head(feats_list, mlp_list, head_list):
    args = list(feats_list) + list(mlp_list) + list(head_list)
    f = pl.pallas_call(
        _head_body,
        out_shape=jax.ShapeDtypeStruct((G, 2), jnp.float32),
    )
    return f(*args)


def kernel(x, edge_index, batch, W1, b1, W2, b2, W3, b3, p1, p2, p3,
           aW1, ab1, aW2, ab2, aW3, ab3, aW4, ab4, aW5, ab5, aW6, ab6,
           lW1, lb1, lW2, lb2, lW3, lb3):
    ei = edge_index.astype(jnp.int32)
    batch_i = batch.astype(jnp.int32)

    pad_e = EP - E
    rows = jnp.concatenate([ei[0], jnp.full((pad_e,), N, jnp.int32)])
    cols = jnp.concatenate([ei[1], jnp.full((pad_e,), N, jnp.int32)])
    rows3 = rows.reshape(NS, NCH, CH)
    cols3 = cols.reshape(NS, NCH, CH)

    xp = jnp.pad(x, ((0, NP - N), (0, 0)))
    h_lo = xp[:, :128]
    h_hi = xp[:, 128:]
    validf = jnp.pad(jnp.ones((N,), jnp.float32), (0, NP - N))
    valid_c = validf.reshape(NP, 1)
    oh = (batch_i[:, None] == jnp.arange(G, dtype=jnp.int32)[None, :])
    oh = jnp.pad(oh.astype(jnp.float32), ((0, NP - N), (0, 0)))
    starts = jnp.searchsorted(batch_i, jnp.arange(G + 1, dtype=jnp.int32),
                              side="left").astype(jnp.int32)

    zeros_np = jnp.zeros((NP,), jnp.float32)
    zeros_half = jnp.zeros((NP, 128), jnp.float32)

    layer_params = [(W1, b1, p1), (W2, b2, p2), (W3, b3, p3)]
    feats = []
    keepf = validf
    for (W, b, p) in layer_params:
        degp = _sc_degree(rows, cols, keepf, zeros_np)
        degp_t = degp.T
        hs_lo, hs_hi, dis_c = _tc_linear(
            h_lo, h_hi, W, b.reshape(1, D), valid_c, degp_t)
        acc_lo, acc_hi = _sc_messages2(rows3, cols3, hs_lo, hs_hi, zeros_half)
        hrel_lo, hrel_hi, score_c = _tc_score(
            acc_lo, acc_hi, hs_lo, hs_hi, dis_c,
            p[:128].reshape(128, 1), p[128:].reshape(128, 1))
        keep_c = _tc_topk(score_c, valid_c, oh)
        hn_lo, hn_hi = _tc_mask(hrel_lo, hrel_hi, keep_c, score_c)
        mlo, mhi, slo, shi, cnt = _tc_feats(starts, hn_lo, hn_hi, keep_c)
        feats.extend([mlo, mhi, slo, shi, cnt])
        h_lo, h_hi = hn_lo, hn_hi
        keepf = keep_c.reshape(NP)
        valid_c = keep_c

    mlp_list = [aW1, ab1.reshape(1, -1), aW2, ab2.reshape(1, -1),
                aW3, ab3.reshape(1, -1), aW4, ab4.reshape(1, -1),
                aW5, ab5.reshape(1, -1), aW6, ab6.reshape(1, -1)]
    head_list = [lW1, lb1.reshape(1, -1), lW2, lb2.reshape(1, -1),
                 lW3, lb3.reshape(1, -1)]
    return _tc_head(feats, mlp_list, head_list)


# speedup vs baseline: 1.1293x; 1.0561x over previous
import functools

import jax
import jax.numpy as jnp
from jax import lax
from jax.experimental import pallas as pl
from jax.experimental.pallas import tpu as pltpu
from jax.experimental.pallas import tpu_sc as plsc

N = 10000
E = 160000
D = 256
G = 64
RATIO = 0.8

NP = 10240
NC, NS, L = 2, 16, 16
NW = NC * NS
EP = 163840
EPW = EP // NW
EPT = EP // NS
CH = 128
NCH = EPT // CH
SLAB = NP // NS

_BIG_NEG = -3.4e38


def _deg_body(rows_hbm, cols_hbm, keepf_hbm, zeros_hbm, out_hbm,
              rows_v, cols_v, keepf_v, deg_v):
    cid = lax.axis_index("c")
    sid = lax.axis_index("s")
    wid = sid * NC + cid
    base = wid * EPW
    pltpu.sync_copy(rows_hbm.at[pl.ds(base, EPW)], rows_v)
    pltpu.sync_copy(cols_hbm.at[pl.ds(base, EPW)], cols_v)
    pltpu.sync_copy(keepf_hbm, keepf_v)
    pltpu.sync_copy(zeros_hbm, deg_v)

    def ebody(i, _):
        idx_c = cols_v[pl.ds(i * L, L)]
        idx_r = rows_v[pl.ds(i * L, L)]
        vals = plsc.load_gather(keepf_v, [idx_c])
        plsc.addupdate_scatter(deg_v, [idx_r], vals)
        return 0

    lax.fori_loop(0, EPW // L, ebody, 0)
    pltpu.sync_copy(deg_v, out_hbm.at[wid])


def _sc_degree(rows, cols, keepf, zeros_np):
    mesh = plsc.VectorSubcoreMesh(core_axis_name="c", subcore_axis_name="s")
    f = pl.kernel(
        _deg_body,
        out_type=jax.ShapeDtypeStruct((NW, NP), jnp.float32),
        mesh=mesh,
        compiler_params=pltpu.CompilerParams(needs_layout_passes=False),
        scratch_types=[
            pltpu.VMEM((EPW,), jnp.int32),
            pltpu.VMEM((EPW,), jnp.int32),
            pltpu.VMEM((NP,), jnp.float32),
            pltpu.VMEM((NP,), jnp.float32),
        ],
    )
    return f(rows, cols, keepf, zeros_np)


RB = 4


def _msg_half(hs_hbm, rows3_hbm, cols3_hbm, out_hbm, sid,
              rring, cring, buf2, acc_sh, gsem, rsem, csem):
    def idx_fetch(g):
        s = lax.rem(g, RB)
        pltpu.make_async_copy(rows3_hbm.at[sid, g], rring.at[s],
                              rsem.at[s]).start()
        pltpu.make_async_copy(cols3_hbm.at[sid, g], cring.at[s],
                              csem.at[s]).start()

    idx_fetch(0)
    idx_fetch(1)

    def chunk(g, _):
        s = lax.rem(g, RB)
        pltpu.make_async_copy(rows3_hbm.at[sid, 0], rring.at[s],
                              rsem.at[s]).wait()
        pltpu.make_async_copy(cols3_hbm.at[sid, 0], cring.at[s],
                              csem.at[s]).wait()
        p = lax.rem(g, 2)
        pltpu.make_async_copy(hs_hbm.at[rring.at[s]], buf2.at[p],
                              gsem.at[p]).start()

        @pl.when(g + 2 < NCH)
        def _():
            idx_fetch(g + 2)

        @pl.when(g > 0)
        def _():
            q = lax.rem(g + 1, 2)
            s1 = lax.rem(g + RB - 1, RB)
            pltpu.make_async_copy(hs_hbm.at[pl.ds(0, CH)], buf2.at[q],
                                  gsem.at[q]).wait()
            pltpu.sync_copy(buf2.at[q], acc_sh.at[cring.at[s1]], add=True)
        return 0

    lax.fori_loop(0, NCH, chunk, 0)
    qf = (NCH - 1) % 2
    sf = (NCH - 1) % RB
    pltpu.make_async_copy(hs_hbm.at[pl.ds(0, CH)], buf2.at[qf],
                          gsem.at[qf]).wait()
    pltpu.sync_copy(buf2.at[qf], acc_sh.at[cring.at[sf]], add=True)
    plsc.subcore_barrier()

    def wb(j, _):
        r0 = sid * SLAB + j * CH
        pltpu.sync_copy(acc_sh.at[pl.ds(r0, CH)], buf2.at[0])
        pltpu.sync_copy(buf2.at[0], out_hbm.at[pl.ds(r0, CH)])
        return 0

    lax.fori_loop(0, SLAB // CH, wb, 0)


def _msg_body2(rows3_hbm, cols3_hbm, hlo_hbm, hhi_hbm, zeros_hbm,
               outlo_hbm, outhi_hbm, rring, cring, buf2, acc_sh,
               gsem, rsem, csem):
    cid = lax.axis_index("c")
    sid = lax.axis_index("s")
    r0 = sid * SLAB
    pltpu.sync_copy(zeros_hbm.at[pl.ds(r0, SLAB)], acc_sh.at[pl.ds(r0, SLAB)])
    plsc.subcore_barrier()

    @pl.when(cid == 0)
    def _():
        _msg_half(hlo_hbm, rows3_hbm, cols3_hbm, outlo_hbm, sid,
                  rring, cring, buf2, acc_sh, gsem, rsem, csem)

    @pl.when(cid == 1)
    def _():
        _msg_half(hhi_hbm, rows3_hbm, cols3_hbm, outhi_hbm, sid,
                  rring, cring, buf2, acc_sh, gsem, rsem, csem)


def _sc_messages2(rows3, cols3, hs_lo, hs_hi, zeros_half):
    mesh = plsc.VectorSubcoreMesh(core_axis_name="c", subcore_axis_name="s")
    f = pl.kernel(
        _msg_body2,
        out_type=(jax.ShapeDtypeStruct((NP, 128), jnp.float32),
                  jax.ShapeDtypeStruct((NP, 128), jnp.float32)),
        mesh=mesh,
        compiler_params=pltpu.CompilerParams(needs_layout_passes=False),
        scratch_types=[
            pltpu.VMEM((RB, CH), jnp.int32),
            pltpu.VMEM((RB, CH), jnp.int32),
            pltpu.VMEM((2, CH, 128), jnp.float32),
            pltpu.VMEM_SHARED((NP, 128), jnp.float32),
            pltpu.SemaphoreType.DMA((2,)),
            pltpu.SemaphoreType.DMA((RB,)),
            pltpu.SemaphoreType.DMA((RB,)),
        ],
    )
    return f(rows3, cols3, hs_lo, hs_hi, zeros_half)


def _lin_body(hlo_ref, hhi_ref, m_ref, W_ref, b_ref, valid_ref, degp_ref,
              hslo_ref, hshi_ref, dis_ref):
    deg = jnp.sum(degp_ref[...], axis=1, keepdims=True) + valid_ref[...]
    dis = valid_ref[...] * lax.rsqrt(jnp.maximum(deg, jnp.float32(1e-30)))
    dis_ref[...] = dis
    W = W_ref[...]
    wlo = W[:, :128]
    whi = W[:, 128:]
    m = m_ref[...]
    hlin = (lax.dot_general(m * hlo_ref[...], wlo, (((1,), (1,)), ((), ())),
                            preferred_element_type=jnp.float32)
            + lax.dot_general(m * hhi_ref[...], whi, (((1,), (1,)), ((), ())),
                              preferred_element_type=jnp.float32)
            + b_ref[...])
    hs = hlin * dis
    hslo_ref[...] = hs[:, :128]
    hshi_ref[...] = hs[:, 128:]


def _tc_linear(h_lo, h_hi, m_col, W, b2, valid_c, degp_t):
    BR = 1024
    nblk = NP // BR
    grid = (nblk,)
    f = pl.pallas_call(
        _lin_body,
        grid=grid,
        in_specs=[
            pl.BlockSpec((BR, 128), lambda i: (i, 0)),
            pl.BlockSpec((BR, 128), lambda i: (i, 0)),
            pl.BlockSpec((BR, 1), lambda i: (i, 0)),
            pl.BlockSpec((D, D), lambda i: (0, 0)),
            pl.BlockSpec((1, D), lambda i: (0, 0)),
            pl.BlockSpec((BR, 1), lambda i: (i, 0)),
            pl.BlockSpec((BR, NW), lambda i: (i, 0)),
        ],
        out_specs=[
            pl.BlockSpec((BR, 128), lambda i: (i, 0)),
            pl.BlockSpec((BR, 128), lambda i: (i, 0)),
            pl.BlockSpec((BR, 1), lambda i: (i, 0)),
        ],
        out_shape=[
            jax.ShapeDtypeStruct((NP, 128), jnp.float32),
            jax.ShapeDtypeStruct((NP, 128), jnp.float32),
            jax.ShapeDtypeStruct((NP, 1), jnp.float32),
        ],
    )
    return f(h_lo, h_hi, m_col, W, b2, valid_c, degp_t)


def _score_body(acclo_ref, acchi_ref, hslo_ref, hshi_ref, dis_ref,
                plo_ref, phi_ref, hrlo_ref, hrhi_ref, score_ref):
    dis = dis_ref[...]
    hrel_lo = jnp.maximum(dis * (acclo_ref[...] + hslo_ref[...]), 0.0)
    hrel_hi = jnp.maximum(dis * (acchi_ref[...] + hshi_ref[...]), 0.0)
    hrlo_ref[...] = hrel_lo
    hrhi_ref[...] = hrel_hi
    plo = plo_ref[...]
    phi = phi_ref[...]
    pnorm = lax.rsqrt(jnp.sum(plo * plo) + jnp.sum(phi * phi))
    sc = (lax.dot_general(plo, hrel_lo, (((0,), (1,)), ((), ())),
                          preferred_element_type=jnp.float32)
          + lax.dot_general(phi, hrel_hi, (((0,), (1,)), ((), ())),
                            preferred_element_type=jnp.float32))
    score_ref[...] = jnp.tanh(sc * pnorm)


def _tc_score(acc_lo, acc_hi, hs_lo, hs_hi, dis_c, p_lo, p_hi):
    BR = 2048
    f = pl.pallas_call(
        _score_body,
        grid=(NP // BR,),
        in_specs=[
            pl.BlockSpec((BR, 128), lambda i: (i, 0)),
            pl.BlockSpec((BR, 128), lambda i: (i, 0)),
            pl.BlockSpec((BR, 128), lambda i: (i, 0)),
            pl.BlockSpec((BR, 128), lambda i: (i, 0)),
            pl.BlockSpec((BR, 1), lambda i: (i, 0)),
            pl.BlockSpec((128, 1), lambda i: (0, 0)),
            pl.BlockSpec((128, 1), lambda i: (0, 0)),
        ],
        out_specs=[
            pl.BlockSpec((BR, 128), lambda i: (i, 0)),
            pl.BlockSpec((BR, 128), lambda i: (i, 0)),
            pl.BlockSpec((1, BR), lambda i: (0, i)),
        ],
        out_shape=[
            jax.ShapeDtypeStruct((NP, 128), jnp.float32),
            jax.ShapeDtypeStruct((NP, 128), jnp.float32),
            jax.ShapeDtypeStruct((1, NP), jnp.float32),
        ],
    )
    return f(acc_lo, acc_hi, hs_lo, hs_hi, dis_c, p_lo, p_hi)


def _topk_body(score_ref, valid_ref, oh_ref, keepr_ref, keepc_ref, mcol_ref):
    score = score_ref[...]
    b = lax.bitcast_convert_type(score, jnp.int32)
    minint = jnp.int32(-2147483648)
    key = jnp.where(b < 0, minint - b, b)
    k_hi = ((key >> 16) + 32768).astype(jnp.float32)
    k_lo = (key & 0xFFFF).astype(jnp.float32)

    valid = valid_ref[...]
    oh = oh_ref[...]
    counts = lax.dot_general(valid, oh, (((1,), (0,)), ((), ())),
                             preferred_element_type=jnp.float32)
    kf = jnp.ceil(jnp.float32(RATIO) * counts)

    def accept(pred):
        cnt = lax.dot_general(pred, oh, (((1,), (0,)), ((), ())),
                              preferred_element_type=jnp.float32)
        ok = (cnt >= kf).astype(jnp.float32)
        return lax.dot_general(ok, oh, (((1,), (1,)), ((), ())),
                               preferred_element_type=jnp.float32)

    def hi_step(i, carry):
        t_hi, bitv = carry
        c_hi = t_hi + bitv
        pred = (k_hi >= c_hi).astype(jnp.float32) * valid
        return (t_hi + accept(pred) * bitv, bitv * 0.5)

    def lo_step(i, carry):
        t_hi, t_lo, bitv = carry
        c_lo = t_lo + bitv
        pred = (((k_hi > t_hi) | ((k_hi == t_hi) & (k_lo >= c_lo)))
                .astype(jnp.float32) * valid)
        return (t_hi, t_lo + accept(pred) * bitv, bitv * 0.5)

    t0 = jnp.zeros((1, NP), jnp.float32)
    t_hi, _ = lax.fori_loop(0, 16, hi_step, (t0, jnp.float32(32768.0)))
    _, t_lo, _ = lax.fori_loop(0, 16, lo_step,
                               (t_hi, t0, jnp.float32(32768.0)))
    keep = (((k_hi > t_hi) | ((k_hi == t_hi) & (k_lo >= t_lo)))
            .astype(jnp.float32) * valid)
    keepr_ref[...] = keep
    keepc_ref[...] = keep.reshape(NP, 1)
    mcol_ref[...] = (keep * score).reshape(NP, 1)


def _tc_topk(score_row, valid_row, oh):
    f = pl.pallas_call(
        _topk_body,
        out_shape=[
            jax.ShapeDtypeStruct((1, NP), jnp.float32),
            jax.ShapeDtypeStruct((NP, 1), jnp.float32),
            jax.ShapeDtypeStruct((NP, 1), jnp.float32),
        ],
    )
    return f(score_row, valid_row, oh)


def _feats_body(starts_ref, hrlo_ref, hrhi_ref, keep_ref, m_ref,
                maxlo_ref, maxhi_ref, sumlo_ref, sumhi_ref, cnt_ref):
    g = pl.program_id(0)
    start = starts_ref[g]
    end = starts_ref[g + 1]
    nb = (end - start + 7) // 8

    def body(i, carry):
        mlo, mhi, slo, shi, c = carry
        r0 = start + i * 8
        pos = r0 + lax.broadcasted_iota(jnp.int32, (8, 1), 0)
        inseg = (pos < end).astype(jnp.float32)
        kp = keep_ref[pl.ds(r0, 8), :] * inseg
        mm = m_ref[pl.ds(r0, 8), :] * inseg
        rl = hrlo_ref[pl.ds(r0, 8), :] * mm
        rh = hrhi_ref[pl.ds(r0, 8), :] * mm
        mlo = jnp.maximum(mlo, jnp.where(kp > 0, rl, _BIG_NEG))
        mhi = jnp.maximum(mhi, jnp.where(kp > 0, rh, _BIG_NEG))
        slo = slo + rl * kp
        shi = shi + rh * kp
        c = c + jnp.sum(kp)
        return mlo, mhi, slo, shi, c

    init = (jnp.full((8, 128), _BIG_NEG, jnp.float32),
            jnp.full((8, 128), _BIG_NEG, jnp.float32),
            jnp.zeros((8, 128), jnp.float32),
            jnp.zeros((8, 128), jnp.float32),
            jnp.float32(0.0))
    mlo, mhi, slo, shi, c = lax.fori_loop(0, nb, body, init)
    maxlo_ref[...] = jnp.broadcast_to(
        jnp.max(mlo, axis=0, keepdims=True), (8, 128)).reshape(1, 8, 128)
    maxhi_ref[...] = jnp.broadcast_to(
        jnp.max(mhi, axis=0, keepdims=True), (8, 128)).reshape(1, 8, 128)
    sumlo_ref[...] = jnp.broadcast_to(
        jnp.sum(slo, axis=0, keepdims=True), (8, 128)).reshape(1, 8, 128)
    sumhi_ref[...] = jnp.broadcast_to(
        jnp.sum(shi, axis=0, keepdims=True), (8, 128)).reshape(1, 8, 128)
    cnt_ref[...] = jnp.full((1, 8, 1), c, jnp.float32)


def _tc_feats(starts, hrel_lo, hrel_hi, keep_c, m_col):
    grid_spec = pltpu.PrefetchScalarGridSpec(
        num_scalar_prefetch=1,
        grid=(G,),
        in_specs=[
            pl.BlockSpec((NP, 128), lambda g, s: (0, 0)),
            pl.BlockSpec((NP, 128), lambda g, s: (0, 0)),
            pl.BlockSpec((NP, 1), lambda g, s: (0, 0)),
            pl.BlockSpec((NP, 1), lambda g, s: (0, 0)),
        ],
        out_specs=[
            pl.BlockSpec((1, 8, 128), lambda g, s: (g, 0, 0)),
            pl.BlockSpec((1, 8, 128), lambda g, s: (g, 0, 0)),
            pl.BlockSpec((1, 8, 128), lambda g, s: (g, 0, 0)),
            pl.BlockSpec((1, 8, 128), lambda g, s: (g, 0, 0)),
            pl.BlockSpec((1, 8, 1), lambda g, s: (g, 0, 0)),
        ],
    )
    f = pl.pallas_call(
        _feats_body,
        grid_spec=grid_spec,
        out_shape=[
            jax.ShapeDtypeStruct((G, 8, 128), jnp.float32),
            jax.ShapeDtypeStruct((G, 8, 128), jnp.float32),
            jax.ShapeDtypeStruct((G, 8, 128), jnp.float32),
            jax.ShapeDtypeStruct((G, 8, 128), jnp.float32),
            jax.ShapeDtypeStruct((G, 8, 1), jnp.float32),
        ],
    )
    mlo, mhi, slo, shi, cnt = f(starts, hrel_lo, hrel_hi, keep_c, m_col)
    return mlo[:, 0, :], mhi[:, 0, :], slo[:, 0, :], shi[:, 0, :], cnt[:, 0, :]


def _head_body(*refs):
    feats_refs = refs[0:15]
    mlp_refs = refs[15:27]
    head_refs = refs[27:33]
    out_ref = refs[33]

    sx = None
    for li in range(3):
        mlo, mhi, slo, shi, cnt = feats_refs[li * 5:(li + 1) * 5]
        cdiv = jnp.maximum(cnt[...], 1.0)
        f = jnp.concatenate(
            [mlo[...], mhi[...], slo[...] / cdiv, shi[...] / cdiv], axis=1)
        sx = f if sx is None else sx + f
    sx = sx * jnp.float32(1.0 / 3.0)

    a = sx
    for li in range(6):
        W = mlp_refs[li * 2][...]
        b = mlp_refs[li * 2 + 1][...]
        a = jnp.maximum(
            lax.dot_general(a, W, (((1,), (1,)), ((), ())),
                            preferred_element_type=jnp.float32) + b, 0.0)
    z = (a + 1.0) * sx
    for li in range(3):
        W = head_refs[li * 2][...]
        b = head_refs[li * 2 + 1][...]
        z = lax.dot_general(z, W, (((1,), (1,)), ((), ())),
                            preferred_element_type=jnp.float32) + b
        if li < 2:
            z = jnp.maximum(z, 0.0)
    m = jnp.max(z, axis=1, keepdims=True)
    e = z - m
    lse = jnp.log(jnp.sum(jnp.exp(e), axis=1, keepdims=True))
    out_ref[...] = e - lse


def _tc_head(feats_list, mlp_list, head_list):
    args = list(feats_list) + list(mlp_list) + list(head_list)
    f = pl.pallas_call(
        _head_body,
        out_shape=jax.ShapeDtypeStruct((G, 2), jnp.float32),
    )
    return f(*args)


def kernel(x, edge_index, batch, W1, b1, W2, b2, W3, b3, p1, p2, p3,
           aW1, ab1, aW2, ab2, aW3, ab3, aW4, ab4, aW5, ab5, aW6, ab6,
           lW1, lb1, lW2, lb2, lW3, lb3):
    ei = edge_index.astype(jnp.int32)
    batch_i = batch.astype(jnp.int32)

    pad_e = EP - E
    rows = jnp.concatenate([ei[0], jnp.full((pad_e,), N, jnp.int32)])
    cols = jnp.concatenate([ei[1], jnp.full((pad_e,), N, jnp.int32)])
    rows3 = rows.reshape(NS, NCH, CH)
    cols3 = cols.reshape(NS, NCH, CH)

    xp = jnp.pad(x, ((0, NP - N), (0, 0)))
    h_lo = xp[:, :128]
    h_hi = xp[:, 128:]
    validf = jnp.pad(jnp.ones((N,), jnp.float32), (0, NP - N))
    valid_c = validf.reshape(NP, 1)
    oh = (batch_i[:, None] == jnp.arange(G, dtype=jnp.int32)[None, :])
    oh = jnp.pad(oh.astype(jnp.float32), ((0, NP - N), (0, 0)))
    starts = jnp.searchsorted(batch_i, jnp.arange(G + 1, dtype=jnp.int32),
                              side="left").astype(jnp.int32)

    zeros_np = jnp.zeros((NP,), jnp.float32)
    zeros_half = jnp.zeros((NP, 128), jnp.float32)

    layer_params = [(W1, b1, p1), (W2, b2, p2), (W3, b3, p3)]
    feats = []
    keepf = validf
    valid_row = validf.reshape(1, NP)
    m_col = jnp.ones((NP, 1), jnp.float32)
    for (W, b, p) in layer_params:
        degp = _sc_degree(rows, cols, keepf, zeros_np)
        degp_t = degp.T
        hs_lo, hs_hi, dis_c = _tc_linear(
            h_lo, h_hi, m_col, W, b.reshape(1, D), valid_c, degp_t)
        acc_lo, acc_hi = _sc_messages2(rows3, cols3, hs_lo, hs_hi, zeros_half)
        hrel_lo, hrel_hi, score_row = _tc_score(
            acc_lo, acc_hi, hs_lo, hs_hi, dis_c,
            p[:128].reshape(128, 1), p[128:].reshape(128, 1))
        keep_row, keep_c, m_col = _tc_topk(score_row, valid_row, oh)
        mlo, mhi, slo, shi, cnt = _tc_feats(
            starts, hrel_lo, hrel_hi, keep_c, m_col)
        feats.extend([mlo, mhi, slo, shi, cnt])
        h_lo, h_hi = hrel_lo, hrel_hi
        keepf = keep_c.reshape(NP)
        valid_c = keep_c
        valid_row = keep_row

    mlp_list = [aW1, ab1.reshape(1, -1), aW2, ab2.reshape(1, -1),
                aW3, ab3.reshape(1, -1), aW4, ab4.reshape(1, -1),
                aW5, ab5.reshape(1, -1), aW6, ab6.reshape(1, -1)]
    head_list = [lW1, lb1.reshape(1, -1), lW2, lb2.reshape(1, -1),
                 lW3, lb3.reshape(1, -1)]
    return _tc_head(feats, mlp_list, head_list)


# speedup vs baseline: 1.1300x; 1.0006x over previous
import functools

import jax
import jax.numpy as jnp
from jax import lax
from jax.experimental import pallas as pl
from jax.experimental.pallas import tpu as pltpu
from jax.experimental.pallas import tpu_sc as plsc

N = 10000
E = 160000
D = 256
G = 64
RATIO = 0.8

NP = 10240
NC, NS, L = 2, 16, 16
NW = NC * NS
EP = 163840
EPW = EP // NW
EPT = EP // NS
CH = 128
NCH = EPT // CH
SLAB = NP // NS

_BIG_NEG = -3.4e38


def _deg_body(rows_hbm, cols_hbm, keepf_hbm, zeros_hbm, out_hbm,
              rows_v, cols_v, keepf_v, deg_v):
    cid = lax.axis_index("c")
    sid = lax.axis_index("s")
    wid = sid * NC + cid
    base = wid * EPW
    pltpu.sync_copy(rows_hbm.at[pl.ds(base, EPW)], rows_v)
    pltpu.sync_copy(cols_hbm.at[pl.ds(base, EPW)], cols_v)
    pltpu.sync_copy(keepf_hbm, keepf_v)
    pltpu.sync_copy(zeros_hbm, deg_v)

    def ebody(i, _):
        idx_c = cols_v[pl.ds(i * L, L)]
        idx_r = rows_v[pl.ds(i * L, L)]
        vals = plsc.load_gather(keepf_v, [idx_c])
        plsc.addupdate_scatter(deg_v, [idx_r], vals)
        return 0

    lax.fori_loop(0, EPW // L, ebody, 0)
    pltpu.sync_copy(deg_v, out_hbm.at[wid])


def _sc_degree(rows, cols, keepf, zeros_np):
    mesh = plsc.VectorSubcoreMesh(core_axis_name="c", subcore_axis_name="s")
    f = pl.kernel(
        _deg_body,
        out_type=jax.ShapeDtypeStruct((NW, NP), jnp.float32),
        mesh=mesh,
        compiler_params=pltpu.CompilerParams(needs_layout_passes=False),
        scratch_types=[
            pltpu.VMEM((EPW,), jnp.int32),
            pltpu.VMEM((EPW,), jnp.int32),
            pltpu.VMEM((NP,), jnp.float32),
            pltpu.VMEM((NP,), jnp.float32),
        ],
    )
    return f(rows, cols, keepf, zeros_np)


RB = 4


def _msg_half(hs_hbm, rows3_hbm, cols3_hbm, out_hbm, sid,
              rring, cring, buf2, acc_sh, gsem, rsem, csem, ssem):
    def idx_fetch(g):
        s = lax.rem(g, RB)
        pltpu.make_async_copy(rows3_hbm.at[sid, g], rring.at[s],
                              rsem.at[s]).start()
        pltpu.make_async_copy(cols3_hbm.at[sid, g], cring.at[s],
                              csem.at[s]).start()

    idx_fetch(0)
    idx_fetch(1)

    def chunk(g, _):
        @pl.when(g < NCH)
        def _():
            p = lax.rem(g, 2)

            @pl.when(g >= 2)
            def _():
                pltpu.make_async_copy(hs_hbm.at[pl.ds(0, CH)], buf2.at[p],
                                      ssem.at[p]).wait()

            s = lax.rem(g, RB)
            pltpu.make_async_copy(rows3_hbm.at[sid, 0], rring.at[s],
                                  rsem.at[s]).wait()
            pltpu.make_async_copy(hs_hbm.at[rring.at[s]], buf2.at[p],
                                  gsem.at[p]).start()

            @pl.when(g + 2 < NCH)
            def _():
                idx_fetch(g + 2)

        @pl.when(g > 0)
        def _():
            q = lax.rem(g + 1, 2)
            s1 = lax.rem(g + RB - 1, RB)
            pltpu.make_async_copy(hs_hbm.at[pl.ds(0, CH)], buf2.at[q],
                                  gsem.at[q]).wait()
            pltpu.make_async_copy(cols3_hbm.at[sid, 0], cring.at[s1],
                                  csem.at[s1]).wait()
            pltpu.make_async_copy(buf2.at[q], acc_sh.at[cring.at[s1]],
                                  ssem.at[q]).start(add=True)
        return 0

    lax.fori_loop(0, NCH + 1, chunk, 0)
    pltpu.make_async_copy(hs_hbm.at[pl.ds(0, CH)], buf2.at[0],
                          ssem.at[0]).wait()
    pltpu.make_async_copy(hs_hbm.at[pl.ds(0, CH)], buf2.at[1],
                          ssem.at[1]).wait()
    plsc.subcore_barrier()

    def wb(j, _):
        r0 = sid * SLAB + j * CH
        pltpu.sync_copy(acc_sh.at[pl.ds(r0, CH)], buf2.at[0])
        pltpu.sync_copy(buf2.at[0], out_hbm.at[pl.ds(r0, CH)])
        return 0

    lax.fori_loop(0, SLAB // CH, wb, 0)


def _msg_body2(rows3_hbm, cols3_hbm, hlo_hbm, hhi_hbm, zeros_hbm,
               outlo_hbm, outhi_hbm, rring, cring, buf2, acc_sh,
               gsem, rsem, csem, ssem):
    cid = lax.axis_index("c")
    sid = lax.axis_index("s")
    r0 = sid * SLAB
    pltpu.sync_copy(zeros_hbm.at[pl.ds(r0, SLAB)], acc_sh.at[pl.ds(r0, SLAB)])
    plsc.subcore_barrier()

    @pl.when(cid == 0)
    def _():
        _msg_half(hlo_hbm, rows3_hbm, cols3_hbm, outlo_hbm, sid,
                  rring, cring, buf2, acc_sh, gsem, rsem, csem, ssem)

    @pl.when(cid == 1)
    def _():
        _msg_half(hhi_hbm, rows3_hbm, cols3_hbm, outhi_hbm, sid,
                  rring, cring, buf2, acc_sh, gsem, rsem, csem, ssem)


def _sc_messages2(rows3, cols3, hs_lo, hs_hi, zeros_half):
    mesh = plsc.VectorSubcoreMesh(core_axis_name="c", subcore_axis_name="s")
    f = pl.kernel(
        _msg_body2,
        out_type=(jax.ShapeDtypeStruct((NP, 128), jnp.float32),
                  jax.ShapeDtypeStruct((NP, 128), jnp.float32)),
        mesh=mesh,
        compiler_params=pltpu.CompilerParams(needs_layout_passes=False),
        scratch_types=[
            pltpu.VMEM((RB, CH), jnp.int32),
            pltpu.VMEM((RB, CH), jnp.int32),
            pltpu.VMEM((2, CH, 128), jnp.float32),
            pltpu.VMEM_SHARED((NP, 128), jnp.float32),
            pltpu.SemaphoreType.DMA((2,)),
            pltpu.SemaphoreType.DMA((RB,)),
            pltpu.SemaphoreType.DMA((RB,)),
            pltpu.SemaphoreType.DMA((2,)),
        ],
    )
    return f(rows3, cols3, hs_lo, hs_hi, zeros_half)


def _lin_body(hlo_ref, hhi_ref, m_ref, W_ref, b_ref, valid_ref, degp_ref,
              hslo_ref, hshi_ref, dis_ref):
    deg = jnp.sum(degp_ref[...], axis=1, keepdims=True) + valid_ref[...]
    dis = valid_ref[...] * lax.rsqrt(jnp.maximum(deg, jnp.float32(1e-30)))
    dis_ref[...] = dis
    W = W_ref[...]
    wlo = W[:, :128]
    whi = W[:, 128:]
    m = m_ref[...]
    hlin = (lax.dot_general(m * hlo_ref[...], wlo, (((1,), (1,)), ((), ())),
                            preferred_element_type=jnp.float32)
            + lax.dot_general(m * hhi_ref[...], whi, (((1,), (1,)), ((), ())),
                              preferred_element_type=jnp.float32)
            + b_ref[...])
    hs = hlin * dis
    hslo_ref[...] = hs[:, :128]
    hshi_ref[...] = hs[:, 128:]


def _tc_linear(h_lo, h_hi, m_col, W, b2, valid_c, degp_t):
    BR = 1024
    nblk = NP // BR
    grid = (nblk,)
    f = pl.pallas_call(
        _lin_body,
        grid=grid,
        in_specs=[
            pl.BlockSpec((BR, 128), lambda i: (i, 0)),
            pl.BlockSpec((BR, 128), lambda i: (i, 0)),
            pl.BlockSpec((BR, 1), lambda i: (i, 0)),
            pl.BlockSpec((D, D), lambda i: (0, 0)),
            pl.BlockSpec((1, D), lambda i: (0, 0)),
            pl.BlockSpec((BR, 1), lambda i: (i, 0)),
            pl.BlockSpec((BR, NW), lambda i: (i, 0)),
        ],
        out_specs=[
            pl.BlockSpec((BR, 128), lambda i: (i, 0)),
            pl.BlockSpec((BR, 128), lambda i: (i, 0)),
            pl.BlockSpec((BR, 1), lambda i: (i, 0)),
        ],
        out_shape=[
            jax.ShapeDtypeStruct((NP, 128), jnp.float32),
            jax.ShapeDtypeStruct((NP, 128), jnp.float32),
            jax.ShapeDtypeStruct((NP, 1), jnp.float32),
        ],
    )
    return f(h_lo, h_hi, m_col, W, b2, valid_c, degp_t)


def _score_body(acclo_ref, acchi_ref, hslo_ref, hshi_ref, dis_ref,
                plo_ref, phi_ref, hrlo_ref, hrhi_ref, score_ref):
    dis = dis_ref[...]
    hrel_lo = jnp.maximum(dis * (acclo_ref[...] + hslo_ref[...]), 0.0)
    hrel_hi = jnp.maximum(dis * (acchi_ref[...] + hshi_ref[...]), 0.0)
    hrlo_ref[...] = hrel_lo
    hrhi_ref[...] = hrel_hi
    plo = plo_ref[...]
    phi = phi_ref[...]
    pnorm = lax.rsqrt(jnp.sum(plo * plo) + jnp.sum(phi * phi))
    sc = (lax.dot_general(plo, hrel_lo, (((0,), (1,)), ((), ())),
                          preferred_element_type=jnp.float32)
          + lax.dot_general(phi, hrel_hi, (((0,), (1,)), ((), ())),
                            preferred_element_type=jnp.float32))
    score_ref[...] = jnp.tanh(sc * pnorm)


def _tc_score(acc_lo, acc_hi, hs_lo, hs_hi, dis_c, p_lo, p_hi):
    BR = 2048
    f = pl.pallas_call(
        _score_body,
        grid=(NP // BR,),
        in_specs=[
            pl.BlockSpec((BR, 128), lambda i: (i, 0)),
            pl.BlockSpec((BR, 128), lambda i: (i, 0)),
            pl.BlockSpec((BR, 128), lambda i: (i, 0)),
            pl.BlockSpec((BR, 128), lambda i: (i, 0)),
            pl.BlockSpec((BR, 1), lambda i: (i, 0)),
            pl.BlockSpec((128, 1), lambda i: (0, 0)),
            pl.BlockSpec((128, 1), lambda i: (0, 0)),
        ],
        out_specs=[
            pl.BlockSpec((BR, 128), lambda i: (i, 0)),
            pl.BlockSpec((BR, 128), lambda i: (i, 0)),
            pl.BlockSpec((1, BR), lambda i: (0, i)),
        ],
        out_shape=[
            jax.ShapeDtypeStruct((NP, 128), jnp.float32),
            jax.ShapeDtypeStruct((NP, 128), jnp.float32),
            jax.ShapeDtypeStruct((1, NP), jnp.float32),
        ],
    )
    return f(acc_lo, acc_hi, hs_lo, hs_hi, dis_c, p_lo, p_hi)


def _topk_body(score_ref, valid_ref, oh_ref, keepr_ref, keepc_ref, mcol_ref):
    score = score_ref[...]
    b = lax.bitcast_convert_type(score, jnp.int32)
    minint = jnp.int32(-2147483648)
    key = jnp.where(b < 0, minint - b, b)
    k_hi = ((key >> 16) + 32768).astype(jnp.float32)
    k_lo = (key & 0xFFFF).astype(jnp.float32)

    valid = valid_ref[...]
    oh = oh_ref[...]
    counts = lax.dot_general(valid, oh, (((1,), (0,)), ((), ())),
                             preferred_element_type=jnp.float32)
    kf = jnp.ceil(jnp.float32(RATIO) * counts)

    def accept(pred):
        cnt = lax.dot_general(pred, oh, (((1,), (0,)), ((), ())),
                              preferred_element_type=jnp.float32)
        ok = (cnt >= kf).astype(jnp.float32)
        return lax.dot_general(ok, oh, (((1,), (1,)), ((), ())),
                               preferred_element_type=jnp.float32)

    def hi_step(i, carry):
        t_hi, bitv = carry
        c_hi = t_hi + bitv
        pred = (k_hi >= c_hi).astype(jnp.float32) * valid
        return (t_hi + accept(pred) * bitv, bitv * 0.5)

    def lo_step(i, carry):
        t_hi, t_lo, bitv = carry
        c_lo = t_lo + bitv
        pred = (((k_hi > t_hi) | ((k_hi == t_hi) & (k_lo >= c_lo)))
                .astype(jnp.float32) * valid)
        return (t_hi, t_lo + accept(pred) * bitv, bitv * 0.5)

    t0 = jnp.zeros((1, NP), jnp.float32)
    t_hi, _ = lax.fori_loop(0, 16, hi_step, (t0, jnp.float32(32768.0)))
    _, t_lo, _ = lax.fori_loop(0, 16, lo_step,
                               (t_hi, t0, jnp.float32(32768.0)))
    keep = (((k_hi > t_hi) | ((k_hi == t_hi) & (k_lo >= t_lo)))
            .astype(jnp.float32) * valid)
    keepr_ref[...] = keep
    keepc_ref[...] = keep.reshape(NP, 1)
    mcol_ref[...] = (keep * score).reshape(NP, 1)


def _tc_topk(score_row, valid_row, oh):
    f = pl.pallas_call(
        _topk_body,
        out_shape=[
            jax.ShapeDtypeStruct((1, NP), jnp.float32),
            jax.ShapeDtypeStruct((NP, 1), jnp.float32),
            jax.ShapeDtypeStruct((NP, 1), jnp.float32),
        ],
    )
    return f(score_row, valid_row, oh)


def _feats_body(starts_ref, hrlo_ref, hrhi_ref, keep_ref, m_ref,
                maxlo_ref, maxhi_ref, sumlo_ref, sumhi_ref, cnt_ref):
    g = pl.program_id(0)
    start = starts_ref[g]
    end = starts_ref[g + 1]
    nb = (end - start + 7) // 8

    def body(i, carry):
        mlo, mhi, slo, shi, c = carry
        r0 = start + i * 8
        pos = r0 + lax.broadcasted_iota(jnp.int32, (8, 1), 0)
        inseg = (pos < end).astype(jnp.float32)
        kp = keep_ref[pl.ds(r0, 8), :] * inseg
        mm = m_ref[pl.ds(r0, 8), :] * inseg
        rl = hrlo_ref[pl.ds(r0, 8), :] * mm
        rh = hrhi_ref[pl.ds(r0, 8), :] * mm
        mlo = jnp.maximum(mlo, jnp.where(kp > 0, rl, _BIG_NEG))
        mhi = jnp.maximum(mhi, jnp.where(kp > 0, rh, _BIG_NEG))
        slo = slo + rl * kp
        shi = shi + rh * kp
        c = c + jnp.sum(kp)
        return mlo, mhi, slo, shi, c

    init = (jnp.full((8, 128), _BIG_NEG, jnp.float32),
            jnp.full((8, 128), _BIG_NEG, jnp.float32),
            jnp.zeros((8, 128), jnp.float32),
            jnp.zeros((8, 128), jnp.float32),
            jnp.float32(0.0))
    mlo, mhi, slo, shi, c = lax.fori_loop(0, nb, body, init)
    maxlo_ref[...] = jnp.broadcast_to(
        jnp.max(mlo, axis=0, keepdims=True), (8, 128)).reshape(1, 8, 128)
    maxhi_ref[...] = jnp.broadcast_to(
        jnp.max(mhi, axis=0, keepdims=True), (8, 128)).reshape(1, 8, 128)
    sumlo_ref[...] = jnp.broadcast_to(
        jnp.sum(slo, axis=0, keepdims=True), (8, 128)).reshape(1, 8, 128)
    sumhi_ref[...] = jnp.broadcast_to(
        jnp.sum(shi, axis=0, keepdims=True), (8, 128)).reshape(1, 8, 128)
    cnt_ref[...] = jnp.full((1, 8, 1), c, jnp.float32)


def _tc_feats(starts, hrel_lo, hrel_hi, keep_c, m_col):
    grid_spec = pltpu.PrefetchScalarGridSpec(
        num_scalar_prefetch=1,
        grid=(G,),
        in_specs=[
            pl.BlockSpec((NP, 128), lambda g, s: (0, 0)),
            pl.BlockSpec((NP, 128), lambda g, s: (0, 0)),
            pl.BlockSpec((NP, 1), lambda g, s: (0, 0)),
            pl.BlockSpec((NP, 1), lambda g, s: (0, 0)),
        ],
        out_specs=[
            pl.BlockSpec((1, 8, 128), lambda g, s: (g, 0, 0)),
            pl.BlockSpec((1, 8, 128), lambda g, s: (g, 0, 0)),
            pl.BlockSpec((1, 8, 128), lambda g, s: (g, 0, 0)),
            pl.BlockSpec((1, 8, 128), lambda g, s: (g, 0, 0)),
            pl.BlockSpec((1, 8, 1), lambda g, s: (g, 0, 0)),
        ],
    )
    f = pl.pallas_call(
        _feats_body,
        grid_spec=grid_spec,
        out_shape=[
            jax.ShapeDtypeStruct((G, 8, 128), jnp.float32),
            jax.ShapeDtypeStruct((G, 8, 128), jnp.float32),
            jax.ShapeDtypeStruct((G, 8, 128), jnp.float32),
            jax.ShapeDtypeStruct((G, 8, 128), jnp.float32),
            jax.ShapeDtypeStruct((G, 8, 1), jnp.float32),
        ],
    )
    mlo, mhi, slo, shi, cnt = f(starts, hrel_lo, hrel_hi, keep_c, m_col)
    return mlo[:, 0, :], mhi[:, 0, :], slo[:, 0, :], shi[:, 0, :], cnt[:, 0, :]


def _head_body(*refs):
    feats_refs = refs[0:15]
    mlp_refs = refs[15:27]
    head_refs = refs[27:33]
    out_ref = refs[33]

    sx = None
    for li in range(3):
        mlo, mhi, slo, shi, cnt = feats_refs[li * 5:(li + 1) * 5]
        cdiv = jnp.maximum(cnt[...], 1.0)
        f = jnp.concatenate(
            [mlo[...], mhi[...], slo[...] / cdiv, shi[...] / cdiv], axis=1)
        sx = f if sx is None else sx + f
    sx = sx * jnp.float32(1.0 / 3.0)

    a = sx
    for li in range(6):
        W = mlp_refs[li * 2][...]
        b = mlp_refs[li * 2 + 1][...]
        a = jnp.maximum(
            lax.dot_general(a, W, (((1,), (1,)), ((), ())),
                            preferred_element_type=jnp.float32) + b, 0.0)
    z = (a + 1.0) * sx
    for li in range(3):
        W = head_refs[li * 2][...]
        b = head_refs[li * 2 + 1][...]
        z = lax.dot_general(z, W, (((1,), (1,)), ((), ())),
                            preferred_element_type=jnp.float32) + b
        if li < 2:
            z = jnp.maximum(z, 0.0)
    m = jnp.max(z, axis=1, keepdims=True)
    e = z - m
    lse = jnp.log(jnp.sum(jnp.exp(e), axis=1, keepdims=True))
    out_ref[...] = e - lse


def _tc_head(feats_list, mlp_list, head_list):
    args = list(feats_list) + list(mlp_list) + list(head_list)
    f = pl.pallas_call(
        _head_body,
        out_shape=jax.ShapeDtypeStruct((G, 2), jnp.float32),
    )
    return f(*args)


def kernel(x, edge_index, batch, W1, b1, W2, b2, W3, b3, p1, p2, p3,
           aW1, ab1, aW2, ab2, aW3, ab3, aW4, ab4, aW5, ab5, aW6, ab6,
           lW1, lb1, lW2, lb2, lW3, lb3):
    ei = edge_index.astype(jnp.int32)
    batch_i = batch.astype(jnp.int32)

    pad_e = EP - E
    rows = jnp.concatenate([ei[0], jnp.full((pad_e,), N, jnp.int32)])
    cols = jnp.concatenate([ei[1], jnp.full((pad_e,), N, jnp.int32)])
    rows3 = rows.reshape(NS, NCH, CH)
    cols3 = cols.reshape(NS, NCH, CH)

    xp = jnp.pad(x, ((0, NP - N), (0, 0)))
    h_lo = xp[:, :128]
    h_hi = xp[:, 128:]
    validf = jnp.pad(jnp.ones((N,), jnp.float32), (0, NP - N))
    valid_c = validf.reshape(NP, 1)
    oh = (batch_i[:, None] == jnp.arange(G, dtype=jnp.int32)[None, :])
    oh = jnp.pad(oh.astype(jnp.float32), ((0, NP - N), (0, 0)))
    starts = jnp.searchsorted(batch_i, jnp.arange(G + 1, dtype=jnp.int32),
                              side="left").astype(jnp.int32)

    zeros_np = jnp.zeros((NP,), jnp.float32)
    zeros_half = jnp.zeros((NP, 128), jnp.float32)

    layer_params = [(W1, b1, p1), (W2, b2, p2), (W3, b3, p3)]
    feats = []
    keepf = validf
    valid_row = validf.reshape(1, NP)
    m_col = jnp.ones((NP, 1), jnp.float32)
    for (W, b, p) in layer_params:
        degp = _sc_degree(rows, cols, keepf, zeros_np)
        degp_t = degp.T
        hs_lo, hs_hi, dis_c = _tc_linear(
            h_lo, h_hi, m_col, W, b.reshape(1, D), valid_c, degp_t)
        acc_lo, acc_hi = _sc_messages2(rows3, cols3, hs_lo, hs_hi, zeros_half)
        hrel_lo, hrel_hi, score_row = _tc_score(
            acc_lo, acc_hi, hs_lo, hs_hi, dis_c,
            p[:128].reshape(128, 1), p[128:].reshape(128, 1))
        keep_row, keep_c, m_col = _tc_topk(score_row, valid_row, oh)
        mlo, mhi, slo, shi, cnt = _tc_feats(
            starts, hrel_lo, hrel_hi, keep_c, m_col)
        feats.extend([mlo, mhi, slo, shi, cnt])
        h_lo, h_hi = hrel_lo, hrel_hi
        keepf = keep_c.reshape(NP)
        valid_c = keep_c
        valid_row = keep_row

    mlp_list = [aW1, ab1.reshape(1, -1), aW2, ab2.reshape(1, -1),
                aW3, ab3.reshape(1, -1), aW4, ab4.reshape(1, -1),
                aW5, ab5.reshape(1, -1), aW6, ab6.reshape(1, -1)]
    head_list = [lW1, lb1.reshape(1, -1), lW2, lb2.reshape(1, -1),
                 lW3, lb3.reshape(1, -1)]
    return _tc_head(feats, mlp_list, head_list)


# speedup vs baseline: 1.3729x; 1.2150x over previous
import functools

import jax
import jax.numpy as jnp
from jax import lax
from jax.experimental import pallas as pl
from jax.experimental.pallas import tpu as pltpu
from jax.experimental.pallas import tpu_sc as plsc

N = 10000
E = 160000
D = 256
G = 64
RATIO = 0.8

NP = 10240
NC, NS, L = 2, 16, 16
NW = NC * NS
EP = 163840
EPW = EP // NW
EPT = EP // NS
CH = 128
NCH = EPT // CH
SLAB = NP // NS

_BIG_NEG = -3.4e38


def _deg_body(rows_hbm, cols_hbm, keepf_hbm, zeros_hbm, out_hbm,
              rows_v, cols_v, keepf_v, deg_v):
    cid = lax.axis_index("c")
    sid = lax.axis_index("s")
    wid = sid * NC + cid
    base = wid * EPW
    pltpu.sync_copy(rows_hbm.at[pl.ds(base, EPW)], rows_v)
    pltpu.sync_copy(cols_hbm.at[pl.ds(base, EPW)], cols_v)
    pltpu.sync_copy(keepf_hbm, keepf_v)
    pltpu.sync_copy(zeros_hbm, deg_v)

    def ebody(i, _):
        idx_c = cols_v[pl.ds(i * L, L)]
        idx_r = rows_v[pl.ds(i * L, L)]
        vals = plsc.load_gather(keepf_v, [idx_c])
        plsc.addupdate_scatter(deg_v, [idx_r], vals)
        return 0

    lax.fori_loop(0, EPW // L, ebody, 0)
    pltpu.sync_copy(deg_v, out_hbm.at[wid])


def _sc_degree(rows, cols, keepf, zeros_np):
    mesh = plsc.VectorSubcoreMesh(core_axis_name="c", subcore_axis_name="s")
    f = pl.kernel(
        _deg_body,
        out_type=jax.ShapeDtypeStruct((NW, NP), jnp.float32),
        mesh=mesh,
        compiler_params=pltpu.CompilerParams(needs_layout_passes=False),
        scratch_types=[
            pltpu.VMEM((EPW,), jnp.int32),
            pltpu.VMEM((EPW,), jnp.int32),
            pltpu.VMEM((NP,), jnp.float32),
            pltpu.VMEM((NP,), jnp.float32),
        ],
    )
    return f(rows, cols, keepf, zeros_np)


RB = 4


def _msg_half(hs_hbm, rows3_hbm, cols3_hbm, out_hbm, sid,
              rring, cring, buf2, acc_sh, gsem, rsem, csem, ssem):
    def idx_fetch(g):
        s = lax.rem(g, RB)
        pltpu.make_async_copy(rows3_hbm.at[sid, g], rring.at[s],
                              rsem.at[s]).start()
        pltpu.make_async_copy(cols3_hbm.at[sid, g], cring.at[s],
                              csem.at[s]).start()

    idx_fetch(0)
    idx_fetch(1)

    def chunk(g, _):
        @pl.when(g < NCH)
        def _():
            p = lax.rem(g, 2)

            @pl.when(g >= 2)
            def _():
                pltpu.make_async_copy(hs_hbm.at[pl.ds(0, CH)], buf2.at[p],
                                      ssem.at[p]).wait()

            s = lax.rem(g, RB)
            pltpu.make_async_copy(rows3_hbm.at[sid, 0], rring.at[s],
                                  rsem.at[s]).wait()
            pltpu.make_async_copy(hs_hbm.at[rring.at[s]], buf2.at[p],
                                  gsem.at[p]).start()

            @pl.when(g + 2 < NCH)
            def _():
                idx_fetch(g + 2)

        @pl.when(g > 0)
        def _():
            q = lax.rem(g + 1, 2)
            s1 = lax.rem(g + RB - 1, RB)
            pltpu.make_async_copy(hs_hbm.at[pl.ds(0, CH)], buf2.at[q],
                                  gsem.at[q]).wait()
            pltpu.make_async_copy(cols3_hbm.at[sid, 0], cring.at[s1],
                                  csem.at[s1]).wait()
            pltpu.make_async_copy(buf2.at[q], acc_sh.at[cring.at[s1]],
                                  ssem.at[q]).start(add=True)
        return 0

    lax.fori_loop(0, NCH + 1, chunk, 0)
    pltpu.make_async_copy(hs_hbm.at[pl.ds(0, CH)], buf2.at[0],
                          ssem.at[0]).wait()
    pltpu.make_async_copy(hs_hbm.at[pl.ds(0, CH)], buf2.at[1],
                          ssem.at[1]).wait()
    plsc.subcore_barrier()

    def wb(j, _):
        r0 = sid * SLAB + j * CH
        pltpu.sync_copy(acc_sh.at[pl.ds(r0, CH)], buf2.at[0])
        pltpu.sync_copy(buf2.at[0], out_hbm.at[pl.ds(r0, CH)])
        return 0

    lax.fori_loop(0, SLAB // CH, wb, 0)


def _msg_body2(rows3_hbm, cols3_hbm, hlo_hbm, hhi_hbm, zeros_hbm,
               outlo_hbm, outhi_hbm, rring, cring, buf2, acc_sh,
               gsem, rsem, csem, ssem):
    cid = lax.axis_index("c")
    sid = lax.axis_index("s")
    r0 = sid * SLAB
    pltpu.sync_copy(zeros_hbm.at[pl.ds(r0, SLAB)], acc_sh.at[pl.ds(r0, SLAB)])
    plsc.subcore_barrier()

    @pl.when(cid == 0)
    def _():
        _msg_half(hlo_hbm, rows3_hbm, cols3_hbm, outlo_hbm, sid,
                  rring, cring, buf2, acc_sh, gsem, rsem, csem, ssem)

    @pl.when(cid == 1)
    def _():
        _msg_half(hhi_hbm, rows3_hbm, cols3_hbm, outhi_hbm, sid,
                  rring, cring, buf2, acc_sh, gsem, rsem, csem, ssem)


def _sc_messages2(rows3, cols3, hs_lo, hs_hi, zeros_half):
    mesh = plsc.VectorSubcoreMesh(core_axis_name="c", subcore_axis_name="s")
    f = pl.kernel(
        _msg_body2,
        out_type=(jax.ShapeDtypeStruct((NP, 128), jnp.float32),
                  jax.ShapeDtypeStruct((NP, 128), jnp.float32)),
        mesh=mesh,
        compiler_params=pltpu.CompilerParams(needs_layout_passes=False),
        scratch_types=[
            pltpu.VMEM((RB, CH), jnp.int32),
            pltpu.VMEM((RB, CH), jnp.int32),
            pltpu.VMEM((2, CH, 128), jnp.float32),
            pltpu.VMEM_SHARED((NP, 128), jnp.float32),
            pltpu.SemaphoreType.DMA((2,)),
            pltpu.SemaphoreType.DMA((RB,)),
            pltpu.SemaphoreType.DMA((RB,)),
            pltpu.SemaphoreType.DMA((2,)),
        ],
    )
    return f(rows3, cols3, hs_lo, hs_hi, zeros_half)


def _lin_body(hlo_ref, hhi_ref, m_ref, W_ref, b_ref, valid_ref, degp_ref,
              hslo_ref, hshi_ref, dis_ref):
    deg = jnp.sum(degp_ref[...], axis=1, keepdims=True) + valid_ref[...]
    dis = valid_ref[...] * lax.rsqrt(jnp.maximum(deg, jnp.float32(1e-30)))
    dis_ref[...] = dis
    W = W_ref[...]
    wlo = W[:, :128]
    whi = W[:, 128:]
    m = m_ref[...]
    hlin = (lax.dot_general(m * hlo_ref[...], wlo, (((1,), (1,)), ((), ())),
                            preferred_element_type=jnp.float32)
            + lax.dot_general(m * hhi_ref[...], whi, (((1,), (1,)), ((), ())),
                              preferred_element_type=jnp.float32)
            + b_ref[...])
    hs = hlin * dis
    hslo_ref[...] = hs[:, :128]
    hshi_ref[...] = hs[:, 128:]


def _tc_linear(h_lo, h_hi, m_col, W, b2, valid_c, degp_t):
    BR = 1024
    nblk = NP // BR
    grid = (nblk,)
    f = pl.pallas_call(
        _lin_body,
        grid=grid,
        in_specs=[
            pl.BlockSpec((BR, 128), lambda i: (i, 0)),
            pl.BlockSpec((BR, 128), lambda i: (i, 0)),
            pl.BlockSpec((BR, 1), lambda i: (i, 0)),
            pl.BlockSpec((D, D), lambda i: (0, 0)),
            pl.BlockSpec((1, D), lambda i: (0, 0)),
            pl.BlockSpec((BR, 1), lambda i: (i, 0)),
            pl.BlockSpec((BR, NW), lambda i: (i, 0)),
        ],
        out_specs=[
            pl.BlockSpec((BR, 128), lambda i: (i, 0)),
            pl.BlockSpec((BR, 128), lambda i: (i, 0)),
            pl.BlockSpec((BR, 1), lambda i: (i, 0)),
        ],
        out_shape=[
            jax.ShapeDtypeStruct((NP, 128), jnp.float32),
            jax.ShapeDtypeStruct((NP, 128), jnp.float32),
            jax.ShapeDtypeStruct((NP, 1), jnp.float32),
        ],
    )
    return f(h_lo, h_hi, m_col, W, b2, valid_c, degp_t)


def _score_body(acclo_ref, acchi_ref, hslo_ref, hshi_ref, dis_ref,
                plo_ref, phi_ref, hrlo_ref, hrhi_ref, score_ref):
    dis = dis_ref[...]
    hrel_lo = jnp.maximum(dis * (acclo_ref[...] + hslo_ref[...]), 0.0)
    hrel_hi = jnp.maximum(dis * (acchi_ref[...] + hshi_ref[...]), 0.0)
    hrlo_ref[...] = hrel_lo
    hrhi_ref[...] = hrel_hi
    plo = plo_ref[...]
    phi = phi_ref[...]
    pnorm = lax.rsqrt(jnp.sum(plo * plo) + jnp.sum(phi * phi))
    sc = (lax.dot_general(plo, hrel_lo, (((0,), (1,)), ((), ())),
                          preferred_element_type=jnp.float32)
          + lax.dot_general(phi, hrel_hi, (((0,), (1,)), ((), ())),
                            preferred_element_type=jnp.float32))
    score_ref[...] = jnp.tanh(sc * pnorm)


def _tc_score(acc_lo, acc_hi, hs_lo, hs_hi, dis_c, p_lo, p_hi):
    BR = 2048
    f = pl.pallas_call(
        _score_body,
        grid=(NP // BR,),
        in_specs=[
            pl.BlockSpec((BR, 128), lambda i: (i, 0)),
            pl.BlockSpec((BR, 128), lambda i: (i, 0)),
            pl.BlockSpec((BR, 128), lambda i: (i, 0)),
            pl.BlockSpec((BR, 128), lambda i: (i, 0)),
            pl.BlockSpec((BR, 1), lambda i: (i, 0)),
            pl.BlockSpec((128, 1), lambda i: (0, 0)),
            pl.BlockSpec((128, 1), lambda i: (0, 0)),
        ],
        out_specs=[
            pl.BlockSpec((BR, 128), lambda i: (i, 0)),
            pl.BlockSpec((BR, 128), lambda i: (i, 0)),
            pl.BlockSpec((1, BR), lambda i: (0, i)),
        ],
        out_shape=[
            jax.ShapeDtypeStruct((NP, 128), jnp.float32),
            jax.ShapeDtypeStruct((NP, 128), jnp.float32),
            jax.ShapeDtypeStruct((1, NP), jnp.float32),
        ],
    )
    return f(acc_lo, acc_hi, hs_lo, hs_hi, dis_c, p_lo, p_hi)


def _topk_body(score_ref, valid_ref, oh_ref, hrlo_ref, hrhi_ref,
               keepr_ref, keepc_ref, mcol_ref, sumlo_ref, sumhi_ref, cnt_ref):
    score = score_ref[...]
    b = lax.bitcast_convert_type(score, jnp.int32)
    minint = jnp.int32(-2147483648)
    key = jnp.where(b < 0, minint - b, b)
    k_hi = ((key >> 16) + 32768).astype(jnp.float32)
    k_lo = (key & 0xFFFF).astype(jnp.float32)

    valid = valid_ref[...]
    oh = oh_ref[...]
    counts = lax.dot_general(valid, oh, (((1,), (0,)), ((), ())),
                             preferred_element_type=jnp.float32)
    kf = jnp.ceil(jnp.float32(RATIO) * counts)

    def accept(pred):
        cnt = lax.dot_general(pred, oh, (((1,), (0,)), ((), ())),
                              preferred_element_type=jnp.float32)
        ok = (cnt >= kf).astype(jnp.float32)
        return lax.dot_general(ok, oh, (((1,), (1,)), ((), ())),
                               preferred_element_type=jnp.float32)

    def hi_step(i, carry):
        t_hi, bitv = carry
        c_hi = t_hi + bitv
        pred = (k_hi >= c_hi).astype(jnp.float32) * valid
        return (t_hi + accept(pred) * bitv, bitv * 0.5)

    def lo_step(i, carry):
        t_hi, t_lo, bitv = carry
        c_lo = t_lo + bitv
        pred = (((k_hi > t_hi) | ((k_hi == t_hi) & (k_lo >= c_lo)))
                .astype(jnp.float32) * valid)
        return (t_hi, t_lo + accept(pred) * bitv, bitv * 0.5)

    t0 = jnp.zeros((1, NP), jnp.float32)
    t_hi, _ = lax.fori_loop(0, 16, hi_step, (t0, jnp.float32(32768.0)))
    _, t_lo, _ = lax.fori_loop(0, 16, lo_step,
                               (t_hi, t0, jnp.float32(32768.0)))
    keep = (((k_hi > t_hi) | ((k_hi == t_hi) & (k_lo >= t_lo)))
            .astype(jnp.float32) * valid)
    keepr_ref[...] = keep
    keepc = keep.reshape(NP, 1)
    mcol = (keep * score).reshape(NP, 1)
    keepc_ref[...] = keepc
    mcol_ref[...] = mcol
    hn_lo = mcol * hrlo_ref[...]
    hn_hi = mcol * hrhi_ref[...]
    sumlo_ref[...] = lax.dot_general(oh, hn_lo, (((0,), (0,)), ((), ())),
                                     preferred_element_type=jnp.float32)
    sumhi_ref[...] = lax.dot_general(oh, hn_hi, (((0,), (0,)), ((), ())),
                                     preferred_element_type=jnp.float32)
    cnt_ref[...] = lax.dot_general(keep, oh, (((1,), (0,)), ((), ())),
                                   preferred_element_type=jnp.float32)


def _tc_topk(score_row, valid_row, oh, hrel_lo, hrel_hi):
    f = pl.pallas_call(
        _topk_body,
        out_shape=[
            jax.ShapeDtypeStruct((1, NP), jnp.float32),
            jax.ShapeDtypeStruct((NP, 1), jnp.float32),
            jax.ShapeDtypeStruct((NP, 1), jnp.float32),
            jax.ShapeDtypeStruct((G, 128), jnp.float32),
            jax.ShapeDtypeStruct((G, 128), jnp.float32),
            jax.ShapeDtypeStruct((1, G), jnp.float32),
        ],
    )
    return f(score_row, valid_row, oh, hrel_lo, hrel_hi)


GPP = 8
BLF = 32


def _feats_body(starts_ref, hrlo_ref, hrhi_ref, keep_ref, m_ref,
                maxlo_ref, maxhi_ref):
    p = pl.program_id(0)

    res_lo = []
    res_hi = []
    for j in range(GPP):
        g = p * GPP + j
        start = starts_ref[g]
        end = starts_ref[g + 1]
        nb = (end - start + BLF - 1) // BLF

        def body(i, carry):
            mlo, mhi = carry
            r0 = start + i * BLF
            pos = r0 + lax.broadcasted_iota(jnp.int32, (BLF, 1), 0)
            inseg = (pos < end).astype(jnp.float32)
            kp = keep_ref[pl.ds(r0, BLF), :] * inseg
            mm = m_ref[pl.ds(r0, BLF), :]
            rl = hrlo_ref[pl.ds(r0, BLF), :] * mm
            rh = hrhi_ref[pl.ds(r0, BLF), :] * mm
            mlo = jnp.maximum(mlo, jnp.where(kp > 0, rl, _BIG_NEG))
            mhi = jnp.maximum(mhi, jnp.where(kp > 0, rh, _BIG_NEG))
            return mlo, mhi

        init = (jnp.full((BLF, 128), _BIG_NEG, jnp.float32),
                jnp.full((BLF, 128), _BIG_NEG, jnp.float32))
        mlo, mhi = lax.fori_loop(0, nb, body, init)
        res_lo.append(jnp.max(mlo, axis=0, keepdims=True))
        res_hi.append(jnp.max(mhi, axis=0, keepdims=True))
    maxlo_ref[...] = jnp.concatenate(res_lo, axis=0)
    maxhi_ref[...] = jnp.concatenate(res_hi, axis=0)


def _tc_feats(starts, hrel_lo, hrel_hi, keep_c, m_col):
    grid_spec = pltpu.PrefetchScalarGridSpec(
        num_scalar_prefetch=1,
        grid=(G // GPP,),
        in_specs=[
            pl.BlockSpec((NP, 128), lambda g, s: (0, 0)),
            pl.BlockSpec((NP, 128), lambda g, s: (0, 0)),
            pl.BlockSpec((NP, 1), lambda g, s: (0, 0)),
            pl.BlockSpec((NP, 1), lambda g, s: (0, 0)),
        ],
        out_specs=[
            pl.BlockSpec((GPP, 128), lambda g, s: (g, 0)),
            pl.BlockSpec((GPP, 128), lambda g, s: (g, 0)),
        ],
    )
    f = pl.pallas_call(
        _feats_body,
        grid_spec=grid_spec,
        out_shape=[
            jax.ShapeDtypeStruct((G, 128), jnp.float32),
            jax.ShapeDtypeStruct((G, 128), jnp.float32),
        ],
    )
    return f(starts, hrel_lo, hrel_hi, keep_c, m_col)


def _head_body(*refs):
    feats_refs = refs[0:15]
    mlp_refs = refs[15:27]
    head_refs = refs[27:33]
    out_ref = refs[33]

    sx = None
    for li in range(3):
        mlo, mhi, slo, shi, cnt = feats_refs[li * 5:(li + 1) * 5]
        cdiv = jnp.maximum(cnt[...], 1.0)
        f = jnp.concatenate(
            [mlo[...], mhi[...], slo[...] / cdiv, shi[...] / cdiv], axis=1)
        sx = f if sx is None else sx + f
    sx = sx * jnp.float32(1.0 / 3.0)

    a = sx
    for li in range(6):
        W = mlp_refs[li * 2][...]
        b = mlp_refs[li * 2 + 1][...]
        a = jnp.maximum(
            lax.dot_general(a, W, (((1,), (1,)), ((), ())),
                            preferred_element_type=jnp.float32) + b, 0.0)
    z = (a + 1.0) * sx
    for li in range(3):
        W = head_refs[li * 2][...]
        b = head_refs[li * 2 + 1][...]
        z = lax.dot_general(z, W, (((1,), (1,)), ((), ())),
                            preferred_element_type=jnp.float32) + b
        if li < 2:
            z = jnp.maximum(z, 0.0)
    m = jnp.max(z, axis=1, keepdims=True)
    e = z - m
    lse = jnp.log(jnp.sum(jnp.exp(e), axis=1, keepdims=True))
    out_ref[...] = e - lse


def _tc_head(feats_list, mlp_list, head_list):
    args = list(feats_list) + list(mlp_list) + list(head_list)
    f = pl.pallas_call(
        _head_body,
        out_shape=jax.ShapeDtypeStruct((G, 2), jnp.float32),
    )
    return f(*args)


def kernel(x, edge_index, batch, W1, b1, W2, b2, W3, b3, p1, p2, p3,
           aW1, ab1, aW2, ab2, aW3, ab3, aW4, ab4, aW5, ab5, aW6, ab6,
           lW1, lb1, lW2, lb2, lW3, lb3):
    ei = edge_index.astype(jnp.int32)
    batch_i = batch.astype(jnp.int32)

    pad_e = EP - E
    rows = jnp.concatenate([ei[0], jnp.full((pad_e,), N, jnp.int32)])
    cols = jnp.concatenate([ei[1], jnp.full((pad_e,), N, jnp.int32)])
    rows3 = rows.reshape(NS, NCH, CH)
    cols3 = cols.reshape(NS, NCH, CH)

    xp = jnp.pad(x, ((0, NP - N), (0, 0)))
    h_lo = xp[:, :128]
    h_hi = xp[:, 128:]
    validf = jnp.pad(jnp.ones((N,), jnp.float32), (0, NP - N))
    valid_c = validf.reshape(NP, 1)
    oh = (batch_i[:, None] == jnp.arange(G, dtype=jnp.int32)[None, :])
    oh = jnp.pad(oh.astype(jnp.float32), ((0, NP - N), (0, 0)))
    starts = jnp.searchsorted(batch_i, jnp.arange(G + 1, dtype=jnp.int32),
                              side="left").astype(jnp.int32)

    zeros_np = jnp.zeros((NP,), jnp.float32)
    zeros_half = jnp.zeros((NP, 128), jnp.float32)

    layer_params = [(W1, b1, p1), (W2, b2, p2), (W3, b3, p3)]
    feats = []
    keepf = validf
    valid_row = validf.reshape(1, NP)
    m_col = jnp.ones((NP, 1), jnp.float32)
    for (W, b, p) in layer_params:
        degp = _sc_degree(rows, cols, keepf, zeros_np)
        degp_t = degp.T
        hs_lo, hs_hi, dis_c = _tc_linear(
            h_lo, h_hi, m_col, W, b.reshape(1, D), valid_c, degp_t)
        acc_lo, acc_hi = _sc_messages2(rows3, cols3, hs_lo, hs_hi, zeros_half)
        hrel_lo, hrel_hi, score_row = _tc_score(
            acc_lo, acc_hi, hs_lo, hs_hi, dis_c,
            p[:128].reshape(128, 1), p[128:].reshape(128, 1))
        keep_row, keep_c, m_col, slo, shi, cnt = _tc_topk(
            score_row, valid_row, oh, hrel_lo, hrel_hi)
        mlo, mhi = _tc_feats(starts, hrel_lo, hrel_hi, keep_c, m_col)
        feats.extend([mlo, mhi, slo, shi, cnt.T])
        h_lo, h_hi = hrel_lo, hrel_hi
        keepf = keep_c.reshape(NP)
        valid_c = keep_c
        valid_row = keep_row

    mlp_list = [aW1, ab1.reshape(1, -1), aW2, ab2.reshape(1, -1),
                aW3, ab3.reshape(1, -1), aW4, ab4.reshape(1, -1),
                aW5, ab5.reshape(1, -1), aW6, ab6.reshape(1, -1)]
    head_list = [lW1, lb1.reshape(1, -1), lW2, lb2.reshape(1, -1),
                 lW3, lb3.reshape(1, -1)]
    return _tc_head(feats, mlp_list, head_list)


# speedup vs baseline: 1.3736x; 1.0005x over previous
import jax
import jax.numpy as jnp
from jax import lax
from jax.experimental import pallas as pl
from jax.experimental.pallas import tpu as pltpu
from jax.experimental.pallas import tpu_sc as plsc

N = 10000
E = 160000
D = 256
G = 64
RATIO = 0.8

NP = 10240
NC, NS, L = 2, 16, 16
NW = NC * NS
EP = 163840
EPW = EP // NW
EPT = EP // NS
CH = 128
NCH = EPT // CH
SLAB = NP // NS

_BIG_NEG = -3.4e38


def _deg_body(rows_hbm, cols_hbm, keepf_hbm, zeros_hbm, out_hbm,
              rows_v, cols_v, keepf_v, deg_v):
    cid = lax.axis_index("c")
    sid = lax.axis_index("s")
    wid = sid * NC + cid
    base = wid * EPW
    pltpu.sync_copy(rows_hbm.at[pl.ds(base, EPW)], rows_v)
    pltpu.sync_copy(cols_hbm.at[pl.ds(base, EPW)], cols_v)
    pltpu.sync_copy(keepf_hbm, keepf_v)
    pltpu.sync_copy(zeros_hbm, deg_v)

    def ebody(i, _):
        idx_c = cols_v[pl.ds(i * L, L)]
        idx_r = rows_v[pl.ds(i * L, L)]
        vals = plsc.load_gather(keepf_v, [idx_c])
        plsc.addupdate_scatter(deg_v, [idx_r], vals)
        return 0

    lax.fori_loop(0, EPW // L, ebody, 0)
    pltpu.sync_copy(deg_v, out_hbm.at[wid])


def _sc_degree(rows, cols, keepf, zeros_np):
    mesh = plsc.VectorSubcoreMesh(core_axis_name="c", subcore_axis_name="s")
    f = pl.kernel(
        _deg_body,
        out_type=jax.ShapeDtypeStruct((NW, NP), jnp.float32),
        mesh=mesh,
        compiler_params=pltpu.CompilerParams(needs_layout_passes=False),
        scratch_types=[
            pltpu.VMEM((EPW,), jnp.int32),
            pltpu.VMEM((EPW,), jnp.int32),
            pltpu.VMEM((NP,), jnp.float32),
            pltpu.VMEM((NP,), jnp.float32),
        ],
    )
    return f(rows, cols, keepf, zeros_np)


RB = 4


def _msg_half(hs_hbm, rows3_hbm, cols3_hbm, out_hbm, sid,
              rring, cring, buf2, acc_sh, gsem, rsem, csem, ssem):
    def idx_fetch(g):
        s = lax.rem(g, RB)
        pltpu.make_async_copy(rows3_hbm.at[sid, g], rring.at[s],
                              rsem.at[s]).start()
        pltpu.make_async_copy(cols3_hbm.at[sid, g], cring.at[s],
                              csem.at[s]).start()

    idx_fetch(0)
    idx_fetch(1)

    def chunk(g, _):
        @pl.when(g < NCH)
        def _():
            p = lax.rem(g, 2)

            @pl.when(g >= 2)
            def _():
                pltpu.make_async_copy(hs_hbm.at[pl.ds(0, CH)], buf2.at[p],
                                      ssem.at[p]).wait()

            s = lax.rem(g, RB)
            pltpu.make_async_copy(rows3_hbm.at[sid, 0], rring.at[s],
                                  rsem.at[s]).wait()
            pltpu.make_async_copy(hs_hbm.at[rring.at[s]], buf2.at[p],
                                  gsem.at[p]).start()

            @pl.when(g + 2 < NCH)
            def _():
                idx_fetch(g + 2)

        @pl.when(g > 0)
        def _():
            q = lax.rem(g + 1, 2)
            s1 = lax.rem(g + RB - 1, RB)
            pltpu.make_async_copy(hs_hbm.at[pl.ds(0, CH)], buf2.at[q],
                                  gsem.at[q]).wait()
            pltpu.make_async_copy(cols3_hbm.at[sid, 0], cring.at[s1],
                                  csem.at[s1]).wait()
            pltpu.make_async_copy(buf2.at[q], acc_sh.at[cring.at[s1]],
                                  ssem.at[q]).start(add=True)
        return 0

    lax.fori_loop(0, NCH + 1, chunk, 0)
    pltpu.make_async_copy(hs_hbm.at[pl.ds(0, CH)], buf2.at[0],
                          ssem.at[0]).wait()
    pltpu.make_async_copy(hs_hbm.at[pl.ds(0, CH)], buf2.at[1],
                          ssem.at[1]).wait()
    plsc.subcore_barrier()
    r0 = sid * SLAB
    cp = pltpu.make_async_copy(acc_sh.at[pl.ds(r0, SLAB)],
                               out_hbm.at[pl.ds(r0, SLAB)], gsem.at[0])
    cp.start()
    cp.wait()


def _msg_body2(rows3_hbm, cols3_hbm, hlo_hbm, hhi_hbm, zeros_hbm,
               outlo_hbm, outhi_hbm, rring, cring, buf2, acc_sh,
               gsem, rsem, csem, ssem):
    cid = lax.axis_index("c")
    sid = lax.axis_index("s")
    r0 = sid * SLAB
    pltpu.sync_copy(zeros_hbm.at[pl.ds(r0, SLAB)], acc_sh.at[pl.ds(r0, SLAB)])
    plsc.subcore_barrier()

    @pl.when(cid == 0)
    def _():
        _msg_half(hlo_hbm, rows3_hbm, cols3_hbm, outlo_hbm, sid,
                  rring, cring, buf2, acc_sh, gsem, rsem, csem, ssem)

    @pl.when(cid == 1)
    def _():
        _msg_half(hhi_hbm, rows3_hbm, cols3_hbm, outhi_hbm, sid,
                  rring, cring, buf2, acc_sh, gsem, rsem, csem, ssem)


def _sc_messages2(rows3, cols3, hs_lo, hs_hi, zeros_half):
    mesh = plsc.VectorSubcoreMesh(core_axis_name="c", subcore_axis_name="s")
    f = pl.kernel(
        _msg_body2,
        out_type=(jax.ShapeDtypeStruct((NP, 128), jnp.float32),
                  jax.ShapeDtypeStruct((NP, 128), jnp.float32)),
        mesh=mesh,
        compiler_params=pltpu.CompilerParams(needs_layout_passes=False),
        scratch_types=[
            pltpu.VMEM((RB, CH), jnp.int32),
            pltpu.VMEM((RB, CH), jnp.int32),
            pltpu.VMEM((2, CH, 128), jnp.float32),
            pltpu.VMEM_SHARED((NP, 128), jnp.float32),
            pltpu.SemaphoreType.DMA((2,)),
            pltpu.SemaphoreType.DMA((RB,)),
            pltpu.SemaphoreType.DMA((RB,)),
            pltpu.SemaphoreType.DMA((2,)),
        ],
    )
    return f(rows3, cols3, hs_lo, hs_hi, zeros_half)


def _lin_body(hlo_ref, hhi_ref, m_ref, W_ref, b_ref, valid_ref, degp_ref,
              hslo_ref, hshi_ref, dis_ref):
    deg = jnp.sum(degp_ref[...], axis=1, keepdims=True) + valid_ref[...]
    dis = valid_ref[...] * lax.rsqrt(jnp.maximum(deg, jnp.float32(1e-30)))
    dis_ref[...] = dis
    W = W_ref[...]
    wlo = W[:, :128]
    whi = W[:, 128:]
    m = m_ref[...]
    hlin = (lax.dot_general(m * hlo_ref[...], wlo, (((1,), (1,)), ((), ())),
                            preferred_element_type=jnp.float32)
            + lax.dot_general(m * hhi_ref[...], whi, (((1,), (1,)), ((), ())),
                              preferred_element_type=jnp.float32)
            + b_ref[...])
    hs = hlin * dis
    hslo_ref[...] = hs[:, :128]
    hshi_ref[...] = hs[:, 128:]


def _tc_linear(h_lo, h_hi, m_col, W, b2, valid_c, degp_t):
    BR = 1024
    nblk = NP // BR
    grid = (nblk,)
    f = pl.pallas_call(
        _lin_body,
        grid=grid,
        in_specs=[
            pl.BlockSpec((BR, 128), lambda i: (i, 0)),
            pl.BlockSpec((BR, 128), lambda i: (i, 0)),
            pl.BlockSpec((BR, 1), lambda i: (i, 0)),
            pl.BlockSpec((D, D), lambda i: (0, 0)),
            pl.BlockSpec((1, D), lambda i: (0, 0)),
            pl.BlockSpec((BR, 1), lambda i: (i, 0)),
            pl.BlockSpec((BR, NW), lambda i: (i, 0)),
        ],
        out_specs=[
            pl.BlockSpec((BR, 128), lambda i: (i, 0)),
            pl.BlockSpec((BR, 128), lambda i: (i, 0)),
            pl.BlockSpec((BR, 1), lambda i: (i, 0)),
        ],
        out_shape=[
            jax.ShapeDtypeStruct((NP, 128), jnp.float32),
            jax.ShapeDtypeStruct((NP, 128), jnp.float32),
            jax.ShapeDtypeStruct((NP, 1), jnp.float32),
        ],
    )
    return f(h_lo, h_hi, m_col, W, b2, valid_c, degp_t)


def _score_body(acclo_ref, acchi_ref, hslo_ref, hshi_ref, dis_ref,
                plo_ref, phi_ref, hrlo_ref, hrhi_ref, score_ref):
    dis = dis_ref[...]
    hrel_lo = jnp.maximum(dis * (acclo_ref[...] + hslo_ref[...]), 0.0)
    hrel_hi = jnp.maximum(dis * (acchi_ref[...] + hshi_ref[...]), 0.0)
    hrlo_ref[...] = hrel_lo
    hrhi_ref[...] = hrel_hi
    plo = plo_ref[...]
    phi = phi_ref[...]
    pnorm = lax.rsqrt(jnp.sum(plo * plo) + jnp.sum(phi * phi))
    sc = (lax.dot_general(plo, hrel_lo, (((0,), (1,)), ((), ())),
                          preferred_element_type=jnp.float32)
          + lax.dot_general(phi, hrel_hi, (((0,), (1,)), ((), ())),
                            preferred_element_type=jnp.float32))
    score_ref[...] = jnp.tanh(sc * pnorm)


def _tc_score(acc_lo, acc_hi, hs_lo, hs_hi, dis_c, p_lo, p_hi):
    BR = 2048
    f = pl.pallas_call(
        _score_body,
        grid=(NP // BR,),
        in_specs=[
            pl.BlockSpec((BR, 128), lambda i: (i, 0)),
            pl.BlockSpec((BR, 128), lambda i: (i, 0)),
            pl.BlockSpec((BR, 128), lambda i: (i, 0)),
            pl.BlockSpec((BR, 128), lambda i: (i, 0)),
            pl.BlockSpec((BR, 1), lambda i: (i, 0)),
            pl.BlockSpec((128, 1), lambda i: (0, 0)),
            pl.BlockSpec((128, 1), lambda i: (0, 0)),
        ],
        out_specs=[
            pl.BlockSpec((BR, 128), lambda i: (i, 0)),
            pl.BlockSpec((BR, 128), lambda i: (i, 0)),
            pl.BlockSpec((1, BR), lambda i: (0, i)),
        ],
        out_shape=[
            jax.ShapeDtypeStruct((NP, 128), jnp.float32),
            jax.ShapeDtypeStruct((NP, 128), jnp.float32),
            jax.ShapeDtypeStruct((1, NP), jnp.float32),
        ],
    )
    return f(acc_lo, acc_hi, hs_lo, hs_hi, dis_c, p_lo, p_hi)


def _topk_body(score_ref, valid_ref, oh_ref, hrlo_ref, hrhi_ref,
               keepr_ref, keepc_ref, mcol_ref, sumlo_ref, sumhi_ref, cnt_ref):
    score = score_ref[...]
    b = lax.bitcast_convert_type(score, jnp.int32)
    minint = jnp.int32(-2147483648)
    key = jnp.where(b < 0, minint - b, b)
    k_hi = ((key >> 16) + 32768).astype(jnp.float32)
    k_lo = (key & 0xFFFF).astype(jnp.float32)

    valid = valid_ref[...]
    oh = oh_ref[...]
    counts = lax.dot_general(valid, oh, (((1,), (0,)), ((), ())),
                             preferred_element_type=jnp.float32)
    kf = jnp.ceil(jnp.float32(RATIO) * counts)

    def accept(pred):
        cnt = lax.dot_general(pred, oh, (((1,), (0,)), ((), ())),
                              preferred_element_type=jnp.float32)
        ok = (cnt >= kf).astype(jnp.float32)
        return lax.dot_general(ok, oh, (((1,), (1,)), ((), ())),
                               preferred_element_type=jnp.float32)

    def hi_step(i, carry):
        t_hi, bitv = carry
        c_hi = t_hi + bitv
        pred = (k_hi >= c_hi).astype(jnp.float32) * valid
        return (t_hi + accept(pred) * bitv, bitv * 0.5)

    def lo_step(i, carry):
        t_hi, t_lo, bitv = carry
        c_lo = t_lo + bitv
        pred = (((k_hi > t_hi) | ((k_hi == t_hi) & (k_lo >= c_lo)))
                .astype(jnp.float32) * valid)
        return (t_hi, t_lo + accept(pred) * bitv, bitv * 0.5)

    t0 = jnp.zeros((1, NP), jnp.float32)
    t_hi, _ = lax.fori_loop(0, 16, hi_step, (t0, jnp.float32(32768.0)))
    _, t_lo, _ = lax.fori_loop(0, 16, lo_step,
                               (t_hi, t0, jnp.float32(32768.0)))
    keep = (((k_hi > t_hi) | ((k_hi == t_hi) & (k_lo >= t_lo)))
            .astype(jnp.float32) * valid)
    keepr_ref[...] = keep
    keepc = keep.reshape(NP, 1)
    mcol = (keep * score).reshape(NP, 1)
    keepc_ref[...] = keepc
    mcol_ref[...] = mcol
    hn_lo = mcol * hrlo_ref[...]
    hn_hi = mcol * hrhi_ref[...]
    sumlo_ref[...] = lax.dot_general(oh, hn_lo, (((0,), (0,)), ((), ())),
                                     preferred_element_type=jnp.float32)
    sumhi_ref[...] = lax.dot_general(oh, hn_hi, (((0,), (0,)), ((), ())),
                                     preferred_element_type=jnp.float32)
    cnt_ref[...] = lax.dot_general(keep, oh, (((1,), (0,)), ((), ())),
                                   preferred_element_type=jnp.float32)


def _tc_topk(score_row, valid_row, oh, hrel_lo, hrel_hi):
    f = pl.pallas_call(
        _topk_body,
        out_shape=[
            jax.ShapeDtypeStruct((1, NP), jnp.float32),
            jax.ShapeDtypeStruct((NP, 1), jnp.float32),
            jax.ShapeDtypeStruct((NP, 1), jnp.float32),
            jax.ShapeDtypeStruct((G, 128), jnp.float32),
            jax.ShapeDtypeStruct((G, 128), jnp.float32),
            jax.ShapeDtypeStruct((1, G), jnp.float32),
        ],
    )
    return f(score_row, valid_row, oh, hrel_lo, hrel_hi)


GPP = 8
BLF = 32


def _feats_body(starts_ref, hrlo_ref, hrhi_ref, keep_ref, m_ref,
                maxlo_ref, maxhi_ref):
    p = pl.program_id(0)

    res_lo = []
    res_hi = []
    for j in range(GPP):
        g = p * GPP + j
        start = starts_ref[g]
        end = starts_ref[g + 1]
        nb = (end - start + BLF - 1) // BLF

        def body(i, carry):
            mlo, mhi = carry
            r0 = start + i * BLF
            pos = r0 + lax.broadcasted_iota(jnp.int32, (BLF, 1), 0)
            inseg = (pos < end).astype(jnp.float32)
            kp = keep_ref[pl.ds(r0, BLF), :] * inseg
            mm = m_ref[pl.ds(r0, BLF), :]
            rl = hrlo_ref[pl.ds(r0, BLF), :] * mm
            rh = hrhi_ref[pl.ds(r0, BLF), :] * mm
            mlo = jnp.maximum(mlo, jnp.where(kp > 0, rl, _BIG_NEG))
            mhi = jnp.maximum(mhi, jnp.where(kp > 0, rh, _BIG_NEG))
            return mlo, mhi

        init = (jnp.full((BLF, 128), _BIG_NEG, jnp.float32),
                jnp.full((BLF, 128), _BIG_NEG, jnp.float32))
        mlo, mhi = lax.fori_loop(0, nb, body, init)
        res_lo.append(jnp.max(mlo, axis=0, keepdims=True))
        res_hi.append(jnp.max(mhi, axis=0, keepdims=True))
    maxlo_ref[...] = jnp.concatenate(res_lo, axis=0)
    maxhi_ref[...] = jnp.concatenate(res_hi, axis=0)


def _tc_feats(starts, hrel_lo, hrel_hi, keep_c, m_col):
    grid_spec = pltpu.PrefetchScalarGridSpec(
        num_scalar_prefetch=1,
        grid=(G // GPP,),
        in_specs=[
            pl.BlockSpec((NP, 128), lambda g, s: (0, 0)),
            pl.BlockSpec((NP, 128), lambda g, s: (0, 0)),
            pl.BlockSpec((NP, 1), lambda g, s: (0, 0)),
            pl.BlockSpec((NP, 1), lambda g, s: (0, 0)),
        ],
        out_specs=[
            pl.BlockSpec((GPP, 128), lambda g, s: (g, 0)),
            pl.BlockSpec((GPP, 128), lambda g, s: (g, 0)),
        ],
    )
    f = pl.pallas_call(
        _feats_body,
        grid_spec=grid_spec,
        out_shape=[
            jax.ShapeDtypeStruct((G, 128), jnp.float32),
            jax.ShapeDtypeStruct((G, 128), jnp.float32),
        ],
    )
    return f(starts, hrel_lo, hrel_hi, keep_c, m_col)


def _head_body(*refs):
    feats_refs = refs[0:15]
    mlp_refs = refs[15:27]
    head_refs = refs[27:33]
    out_ref = refs[33]

    sx = None
    for li in range(3):
        mlo, mhi, slo, shi, cnt = feats_refs[li * 5:(li + 1) * 5]
        cdiv = jnp.maximum(cnt[...], 1.0)
        f = jnp.concatenate(
            [mlo[...], mhi[...], slo[...] / cdiv, shi[...] / cdiv], axis=1)
        sx = f if sx is None else sx + f
    sx = sx * jnp.float32(1.0 / 3.0)

    a = sx
    for li in range(6):
        W = mlp_refs[li * 2][...]
        b = mlp_refs[li * 2 + 1][...]
        a = jnp.maximum(
            lax.dot_general(a, W, (((1,), (1,)), ((), ())),
                            preferred_element_type=jnp.float32) + b, 0.0)
    z = (a + 1.0) * sx
    for li in range(3):
        W = head_refs[li * 2][...]
        b = head_refs[li * 2 + 1][...]
        z = lax.dot_general(z, W, (((1,), (1,)), ((), ())),
                            preferred_element_type=jnp.float32) + b
        if li < 2:
            z = jnp.maximum(z, 0.0)
    m = jnp.max(z, axis=1, keepdims=True)
    e = z - m
    lse = jnp.log(jnp.sum(jnp.exp(e), axis=1, keepdims=True))
    out_ref[...] = e - lse


def _tc_head(feats_list, mlp_list, head_list):
    args = list(feats_list) + list(mlp_list) + list(head_list)
    f = pl.pallas_call(
        _head_body,
        out_shape=jax.ShapeDtypeStruct((G, 2), jnp.float32),
    )
    return f(*args)


def kernel(x, edge_index, batch, W1, b1, W2, b2, W3, b3, p1, p2, p3,
           aW1, ab1, aW2, ab2, aW3, ab3, aW4, ab4, aW5, ab5, aW6, ab6,
           lW1, lb1, lW2, lb2, lW3, lb3):
    ei = edge_index.astype(jnp.int32)
    batch_i = batch.astype(jnp.int32)

    pad_e = EP - E
    rows = jnp.concatenate([ei[0], jnp.full((pad_e,), N, jnp.int32)])
    cols = jnp.concatenate([ei[1], jnp.full((pad_e,), N, jnp.int32)])
    rows3 = rows.reshape(NS, NCH, CH)
    cols3 = cols.reshape(NS, NCH, CH)

    xp = jnp.pad(x, ((0, NP - N), (0, 0)))
    h_lo = xp[:, :128]
    h_hi = xp[:, 128:]
    validf = jnp.pad(jnp.ones((N,), jnp.float32), (0, NP - N))
    valid_c = validf.reshape(NP, 1)
    oh = (batch_i[:, None] == jnp.arange(G, dtype=jnp.int32)[None, :])
    oh = jnp.pad(oh.astype(jnp.float32), ((0, NP - N), (0, 0)))
    starts = jnp.searchsorted(batch_i, jnp.arange(G + 1, dtype=jnp.int32),
                              side="left").astype(jnp.int32)

    zeros_np = jnp.zeros((NP,), jnp.float32)
    zeros_half = jnp.zeros((NP, 128), jnp.float32)

    layer_params = [(W1, b1, p1), (W2, b2, p2), (W3, b3, p3)]
    feats = []
    keepf = validf
    valid_row = validf.reshape(1, NP)
    m_col = jnp.ones((NP, 1), jnp.float32)
    for (W, b, p) in layer_params:
        degp = _sc_degree(rows, cols, keepf, zeros_np)
        degp_t = degp.T
        hs_lo, hs_hi, dis_c = _tc_linear(
            h_lo, h_hi, m_col, W, b.reshape(1, D), valid_c, degp_t)
        acc_lo, acc_hi = _sc_messages2(rows3, cols3, hs_lo, hs_hi, zeros_half)
        hrel_lo, hrel_hi, score_row = _tc_score(
            acc_lo, acc_hi, hs_lo, hs_hi, dis_c,
            p[:128].reshape(128, 1), p[128:].reshape(128, 1))
        keep_row, keep_c, m_col, slo, shi, cnt = _tc_topk(
            score_row, valid_row, oh, hrel_lo, hrel_hi)
        mlo, mhi = _tc_feats(starts, hrel_lo, hrel_hi, keep_c, m_col)
        feats.extend([mlo, mhi, slo, shi, cnt.T])
        h_lo, h_hi = hrel_lo, hrel_hi
        keepf = keep_c.reshape(NP)
        valid_c = keep_c
        valid_row = keep_row

    mlp_list = [aW1, ab1.reshape(1, -1), aW2, ab2.reshape(1, -1),
                aW3, ab3.reshape(1, -1), aW4, ab4.reshape(1, -1),
                aW5, ab5.reshape(1, -1), aW6, ab6.reshape(1, -1)]
    head_list = [lW1, lb1.reshape(1, -1), lW2, lb2.reshape(1, -1),
                 lW3, lb3.reshape(1, -1)]
    return _tc_head(feats, mlp_list, head_list)
